# Initial kernel scaffold; baseline (speedup 1.0000x reference)
import functools

import jax
import jax.numpy as jnp
from jax import lax
from jax.experimental import pallas as pl
from jax.experimental.pallas import tpu as pltpu
from jax.experimental.pallas import tpu_sc as plsc

f32 = jnp.float32
i32 = jnp.int32

NU = 25000
N = 50000
D = 64
E = 800000
B = 1024
HALF = 25088
NP = 2 * HALF
NT = 16
GB = 128
GPT = 391
GRP = 23
NGRP = 17
EP = NT * GPT * GB
ROWS = NT * GPT
STRIPE = HALF // NT
NTRASH = 80

_mesh = plsc.VectorSubcoreMesh(core_axis_name="c", subcore_axis_name="s")


def _prep_body(src_ref, dst_ref, srcg_ref, dl0_ref, dl1_ref, sl0_ref, sl1_ref):
    src = src_ref[...]
    dst = dst_ref[...]
    srcg_ref[...] = src + 88 * (src >= NU).astype(i32)
    r = lax.broadcasted_iota(i32, (ROWS, GB), 0)
    l = lax.broadcasted_iota(i32, (ROWS, GB), 1)
    toff = NU + ((r * GB + l) % NTRASH)
    dl0_ref[...] = jnp.where(dst < NU, dst, toff)
    dl1_ref[...] = jnp.where((dst >= NU) & (dst < N), dst - NU, toff)
    sl0_ref[...] = jnp.where(src < NU, src, toff)
    sl1_ref[...] = jnp.where((src >= NU) & (src < N), src - NU, toff)


_prep = pl.pallas_call(
    _prep_body,
    out_shape=tuple(jax.ShapeDtypeStruct((ROWS, GB), i32) for _ in range(5)),
)


@functools.partial(
    pl.kernel,
    out_type=(
        jax.ShapeDtypeStruct((NP, 16), f32),
        jax.ShapeDtypeStruct((NP, 16), f32),
    ),
    mesh=_mesh,
    scratch_types=[
        pltpu.VMEM((GRP, GB), i32),
        pltpu.VMEM((GRP, GB), i32),
        pltpu.VMEM((GB, 16), f32),
        pltpu.VMEM((STRIPE, 16), f32),
        pltpu.VMEM_SHARED((HALF, 16), f32),
        pltpu.VMEM_SHARED((HALF, 16), f32),
    ],
)
def _deg(srcl_hbm, dloc_hbm, dego_hbm, degi_hbm, sidx, didx, ones_v, zv,
         dego_sh, degi_sh):
    c = lax.axis_index("c")
    s = lax.axis_index("s")

    def fill(i, carry):
        ones_v[i, :] = jnp.ones((16,), f32)
        return carry

    lax.fori_loop(0, GB, fill, None)

    def fillz(i, carry):
        zv[i, :] = jnp.zeros((16,), f32)
        return carry

    lax.fori_loop(0, STRIPE, fillz, None)
    pltpu.sync_copy(zv, dego_sh.at[pl.ds(s * STRIPE, STRIPE)])
    pltpu.sync_copy(zv, degi_sh.at[pl.ds(s * STRIPE, STRIPE)])
    plsc.subcore_barrier()

    def group(g, carry):
        pltpu.sync_copy(srcl_hbm.at[c, s, pl.ds(g * GRP, GRP)], sidx)
        pltpu.sync_copy(dloc_hbm.at[c, s, pl.ds(g * GRP, GRP)], didx)

        def blk(j, carry2):
            pltpu.sync_copy(ones_v, dego_sh.at[sidx.at[j]], add=True)
            pltpu.sync_copy(ones_v, degi_sh.at[didx.at[j]], add=True)
            return carry2

        return lax.fori_loop(0, GRP, blk, carry)

    lax.fori_loop(0, NGRP, group, None)
    plsc.subcore_barrier()
    dst_lo = c * HALF + s * STRIPE
    pltpu.sync_copy(dego_sh.at[pl.ds(s * STRIPE, STRIPE)],
                    dego_hbm.at[pl.ds(dst_lo, STRIPE)])
    pltpu.sync_copy(degi_sh.at[pl.ds(s * STRIPE, STRIPE)],
                    degi_hbm.at[pl.ds(dst_lo, STRIPE)])


@functools.partial(
    pl.kernel,
    out_type=jax.ShapeDtypeStruct((NP, D), f32),
    mesh=_mesh,
    scratch_types=[
        pltpu.VMEM((GRP, GB), i32),
        pltpu.VMEM((GRP, GB), i32),
        pltpu.VMEM((GB, D), f32),
        pltpu.VMEM((196, D), f32),
        pltpu.VMEM_SHARED((HALF, D), f32),
        pltpu.SemaphoreType.DMA,
    ],
)
def _spmm(y_hbm, srcg_hbm, dloc_hbm, z_hbm, sidx, didx, rows, zv, acc_sh, sem):
    c = lax.axis_index("c")
    s = lax.axis_index("s")

    def fillz(i, carry):
        for k in range(4):
            zv[i, pl.ds(k * 16, 16)] = jnp.zeros((16,), f32)
        return carry

    lax.fori_loop(0, 196, fillz, None)
    for q in range(8):
        pltpu.sync_copy(zv, acc_sh.at[pl.ds(s * STRIPE + q * 196, 196)])
    plsc.subcore_barrier()

    def group(g, carry):
        pltpu.sync_copy(srcg_hbm.at[s, pl.ds(g * GRP, GRP)], sidx)
        pltpu.sync_copy(dloc_hbm.at[c, s, pl.ds(g * GRP, GRP)], didx)

        def blk(j, carry2):
            pltpu.async_copy(y_hbm.at[sidx.at[j]], rows, sem).wait()
            pltpu.sync_copy(rows, acc_sh.at[didx.at[j]], add=True)
            return carry2

        return lax.fori_loop(0, GRP, blk, carry)

    lax.fori_loop(0, NGRP, group, None)
    plsc.subcore_barrier()
    pltpu.sync_copy(acc_sh.at[pl.ds(s * STRIPE, STRIPE)],
                    z_hbm.at[pl.ds(c * HALF + s * STRIPE, STRIPE)])


@functools.partial(
    pl.kernel,
    out_type=jax.ShapeDtypeStruct((B, D), f32),
    mesh=_mesh,
    scratch_types=[
        pltpu.VMEM((B // 32,), i32),
        pltpu.VMEM((B // 32, D), f32),
        pltpu.SemaphoreType.DMA,
    ],
)
def _gather_users(acc_hbm, users_hbm, ue_hbm, uidx, urows, sem):
    wid = lax.axis_index("s") * 2 + lax.axis_index("c")
    base = wid * (B // 32)
    pltpu.sync_copy(users_hbm.at[pl.ds(base, B // 32)], uidx)
    pltpu.async_copy(acc_hbm.at[uidx], urows, sem).wait()
    pltpu.sync_copy(urows, ue_hbm.at[pl.ds(base, B // 32)])


def _dinv_body(dego_ref, degi_ref, dinvo_ref, dinvi_ref):
    dinvo_ref[...] = lax.rsqrt(jnp.maximum(dego_ref[...], 1.0))
    dinvi_ref[...] = lax.rsqrt(jnp.maximum(degi_ref[...], 1.0))


_dinv = pl.pallas_call(
    _dinv_body,
    out_shape=tuple(jax.ShapeDtypeStruct((NP, 16), f32) for _ in range(2)),
)


def _scale0_body(x_ref, dinvo_ref, y_ref):
    y_ref[...] = x_ref[...] * dinvo_ref[...][:, :1]


_scale0 = pl.pallas_call(
    _scale0_body,
    grid=(NP // STRIPE,),
    in_specs=[
        pl.BlockSpec((STRIPE, D), lambda i: (i, 0)),
        pl.BlockSpec((STRIPE, 16), lambda i: (i, 0)),
    ],
    out_specs=pl.BlockSpec((STRIPE, D), lambda i: (i, 0)),
    out_shape=jax.ShapeDtypeStruct((NP, D), f32),
)


def _update_body(z_ref, dinvi_ref, dinvo_ref, accp_ref, acc_ref, y_ref):
    x = z_ref[...] * dinvi_ref[...][:, :1]
    acc_ref[...] = accp_ref[...] + x
    y_ref[...] = x * dinvo_ref[...][:, :1]


_update = pl.pallas_call(
    _update_body,
    grid=(NP // STRIPE,),
    in_specs=[
        pl.BlockSpec((STRIPE, D), lambda i: (i, 0)),
        pl.BlockSpec((STRIPE, 16), lambda i: (i, 0)),
        pl.BlockSpec((STRIPE, 16), lambda i: (i, 0)),
        pl.BlockSpec((STRIPE, D), lambda i: (i, 0)),
    ],
    out_specs=tuple(pl.BlockSpec((STRIPE, D), lambda i: (i, 0)) for _ in range(2)),
    out_shape=tuple(jax.ShapeDtypeStruct((NP, D), f32) for _ in range(2)),
)


def _update_last_body(z_ref, dinvi_ref, accp_ref, acc_ref):
    acc_ref[...] = accp_ref[...] + z_ref[...] * dinvi_ref[...][:, :1]


_update_last = pl.pallas_call(
    _update_last_body,
    grid=(NP // STRIPE,),
    in_specs=[
        pl.BlockSpec((STRIPE, D), lambda i: (i, 0)),
        pl.BlockSpec((STRIPE, 16), lambda i: (i, 0)),
        pl.BlockSpec((STRIPE, D), lambda i: (i, 0)),
    ],
    out_specs=pl.BlockSpec((STRIPE, D), lambda i: (i, 0)),
    out_shape=jax.ShapeDtypeStruct((NP, D), f32),
)


BK = 512
NBK = 49


def _matmul_body(ue_ref, items_ref, out_ref):
    out_ref[...] = lax.dot_general(
        ue_ref[...], items_ref[...],
        (((1,), (1,)), ((), ())),
        preferred_element_type=f32,
    ) * (1.0 / 16.0)


_matmul = pl.pallas_call(
    _matmul_body,
    grid=(NBK,),
    in_specs=[
        pl.BlockSpec((B, D), lambda j: (0, 0)),
        pl.BlockSpec((BK, D), lambda j: (j + HALF // BK, 0)),
    ],
    out_specs=pl.BlockSpec((B, BK), lambda j: (0, j)),
    out_shape=jax.ShapeDtypeStruct((B, NU), f32),
)


def kernel(users, edge_index, user_embedding, item_embedding):
    src = edge_index[0].astype(i32)
    dst = edge_index[1].astype(i32)
    padn = EP - E
    fill = (lax.iota(i32, padn) * 131) % N
    srcp = jnp.concatenate([src, fill]).reshape(ROWS, GB)
    dstp = jnp.concatenate([dst, jnp.full((padn,), N, i32)]).reshape(ROWS, GB)

    srcg, dl0, dl1, sl0, sl1 = _prep(srcp, dstp)
    srcg3 = srcg.reshape(NT, GPT, GB)
    dloc4 = jnp.stack([dl0, dl1]).reshape(2, NT, GPT, GB)
    srcl4 = jnp.stack([sl0, sl1]).reshape(2, NT, GPT, GB)

    dego, degi = _deg(srcl4, dloc4)
    dinvo, dinvi = _dinv(dego, degi)

    zpad = jnp.zeros((HALF - NU, D), f32)
    x0 = jnp.concatenate([user_embedding, zpad, item_embedding, zpad], axis=0)
    acc = x0
    y = _scale0(x0, dinvo)
    for t in range(3):
        z = _spmm(y, srcg3, dloc4)
        if t < 2:
            acc, y = _update(z, dinvi, dinvo, acc)
        else:
            acc = _update_last(z, dinvi, acc)

    ue = _gather_users(acc, users.astype(i32))
    return _matmul(ue, acc)


# speedup vs baseline: 9.2046x; 9.2046x over previous
import functools

import jax
import jax.numpy as jnp
from jax import lax
from jax.experimental import pallas as pl
from jax.experimental.pallas import tpu as pltpu
from jax.experimental.pallas import tpu_sc as plsc

f32 = jnp.float32
i32 = jnp.int32

NU = 25000
N = 50000
D = 64
E = 800000
B = 1024
HALF = 25088
NP = 2 * HALF
NT = 16
GB = 128
GPT = 392
GRP = 8
NGRP = 49
EP = NT * GPT * GB
ROWS = NT * GPT
STRIPE = HALF // NT
NTRASH = 80

_mesh = plsc.VectorSubcoreMesh(core_axis_name="c", subcore_axis_name="s")


def _prep_body(src_ref, dst_ref, srcg_ref, dl0_ref, dl1_ref, sl0_ref, sl1_ref):
    src = src_ref[...]
    dst = dst_ref[...]
    srcg_ref[...] = src + 88 * (src >= NU).astype(i32)
    pid = pl.program_id(0)
    r = pid * GPT + lax.broadcasted_iota(i32, (GPT, GB), 0)
    l = lax.broadcasted_iota(i32, (GPT, GB), 1)
    toff = NU + ((r * GB + l) % NTRASH)
    dl0_ref[...] = jnp.where(dst < NU, dst, toff)
    dl1_ref[...] = jnp.where((dst >= NU) & (dst < N), dst - NU, toff)
    sl0_ref[...] = jnp.where(src < NU, src, toff)
    sl1_ref[...] = jnp.where((src >= NU) & (src < N), src - NU, toff)


_prep = pl.pallas_call(
    _prep_body,
    grid=(NT,),
    in_specs=[pl.BlockSpec((GPT, GB), lambda i: (i, 0)) for _ in range(2)],
    out_specs=tuple(pl.BlockSpec((GPT, GB), lambda i: (i, 0)) for _ in range(5)),
    out_shape=tuple(jax.ShapeDtypeStruct((ROWS, GB), i32) for _ in range(5)),
)


@functools.partial(
    pl.kernel,
    out_type=(
        jax.ShapeDtypeStruct((NP, 16), f32),
        jax.ShapeDtypeStruct((NP, 16), f32),
    ),
    mesh=_mesh,
    compiler_params=pltpu.CompilerParams(use_tc_tiling_on_sc=False),
    scratch_types=[
        pltpu.VMEM((GRP, GB), i32),
        pltpu.VMEM((GRP, GB), i32),
        pltpu.VMEM((GB, 16), f32),
        pltpu.VMEM((STRIPE, 16), f32),
        pltpu.VMEM_SHARED((HALF, 16), f32),
        pltpu.VMEM_SHARED((HALF, 16), f32),
    ],
)
def _deg(srcl_hbm, dloc_hbm, dego_hbm, degi_hbm, sidx, didx, ones_v, zv,
         dego_sh, degi_sh):
    c = lax.axis_index("c")
    s = lax.axis_index("s")

    def fill(i, carry):
        ones_v[i, :] = jnp.ones((16,), f32)
        return carry

    lax.fori_loop(0, GB, fill, None)

    def fillz(i, carry):
        zv[i, :] = jnp.zeros((16,), f32)
        return carry

    lax.fori_loop(0, STRIPE, fillz, None)
    pltpu.sync_copy(zv, dego_sh.at[pl.ds(s * STRIPE, STRIPE)])
    pltpu.sync_copy(zv, degi_sh.at[pl.ds(s * STRIPE, STRIPE)])
    plsc.subcore_barrier()

    def group(g, carry):
        pltpu.sync_copy(srcl_hbm.at[c, s, pl.ds(g * GRP, GRP)], sidx)
        pltpu.sync_copy(dloc_hbm.at[c, s, pl.ds(g * GRP, GRP)], didx)

        def blk(j, carry2):
            pltpu.sync_copy(ones_v, dego_sh.at[sidx.at[j]], add=True)
            pltpu.sync_copy(ones_v, degi_sh.at[didx.at[j]], add=True)
            return carry2

        return lax.fori_loop(0, GRP, blk, carry)

    lax.fori_loop(0, NGRP, group, None)
    plsc.subcore_barrier()
    dst_lo = c * HALF + s * STRIPE
    pltpu.sync_copy(dego_sh.at[pl.ds(s * STRIPE, STRIPE)],
                    dego_hbm.at[pl.ds(dst_lo, STRIPE)])
    pltpu.sync_copy(degi_sh.at[pl.ds(s * STRIPE, STRIPE)],
                    degi_hbm.at[pl.ds(dst_lo, STRIPE)])


@functools.partial(
    pl.kernel,
    out_type=jax.ShapeDtypeStruct((NP, D), f32),
    mesh=_mesh,
    compiler_params=pltpu.CompilerParams(use_tc_tiling_on_sc=False),
    scratch_types=[
        pltpu.VMEM((GRP, GB), i32),
        pltpu.VMEM((GRP, GB), i32),
        pltpu.VMEM((GB, D), f32),
        pltpu.VMEM((196, D), f32),
        pltpu.VMEM_SHARED((HALF, D), f32),
        pltpu.SemaphoreType.DMA,
    ],
)
def _spmm(y_hbm, srcg_hbm, dloc_hbm, z_hbm, sidx, didx, rows, zv, acc_sh, sem):
    c = lax.axis_index("c")
    s = lax.axis_index("s")

    def fillz(i, carry):
        for k in range(4):
            zv[i, pl.ds(k * 16, 16)] = jnp.zeros((16,), f32)
        return carry

    lax.fori_loop(0, 196, fillz, None)
    for q in range(8):
        pltpu.sync_copy(zv, acc_sh.at[pl.ds(s * STRIPE + q * 196, 196)])
    plsc.subcore_barrier()

    def group(g, carry):
        pltpu.sync_copy(srcg_hbm.at[s, pl.ds(g * GRP, GRP)], sidx)
        pltpu.sync_copy(dloc_hbm.at[c, s, pl.ds(g * GRP, GRP)], didx)

        def blk(j, carry2):
            pltpu.async_copy(y_hbm.at[sidx.at[j]], rows, sem).wait()
            pltpu.sync_copy(rows, acc_sh.at[didx.at[j]], add=True)
            return carry2

        return lax.fori_loop(0, GRP, blk, carry)

    lax.fori_loop(0, NGRP, group, None)
    plsc.subcore_barrier()
    pltpu.sync_copy(acc_sh.at[pl.ds(s * STRIPE, STRIPE)],
                    z_hbm.at[pl.ds(c * HALF + s * STRIPE, STRIPE)])


@functools.partial(
    pl.kernel,
    out_type=jax.ShapeDtypeStruct((B, D), f32),
    mesh=_mesh,
    compiler_params=pltpu.CompilerParams(use_tc_tiling_on_sc=False),
    scratch_types=[
        pltpu.VMEM((B // 32,), i32),
        pltpu.VMEM((B // 32, D), f32),
        pltpu.SemaphoreType.DMA,
    ],
)
def _gather_users(acc_hbm, users_hbm, ue_hbm, uidx, urows, sem):
    wid = lax.axis_index("s") * 2 + lax.axis_index("c")
    base = wid * (B // 32)
    pltpu.sync_copy(users_hbm.at[pl.ds(base, B // 32)], uidx)
    pltpu.async_copy(acc_hbm.at[uidx], urows, sem).wait()
    pltpu.sync_copy(urows, ue_hbm.at[pl.ds(base, B // 32)])


def _dinv_body(dego_ref, degi_ref, dinvo_ref, dinvi_ref):
    dinvo_ref[...] = lax.rsqrt(jnp.maximum(dego_ref[...], 1.0))
    dinvi_ref[...] = lax.rsqrt(jnp.maximum(degi_ref[...], 1.0))


_dinv = pl.pallas_call(
    _dinv_body,
    grid=(NP // STRIPE,),
    in_specs=[pl.BlockSpec((STRIPE, 16), lambda i: (i, 0)) for _ in range(2)],
    out_specs=tuple(pl.BlockSpec((STRIPE, 16), lambda i: (i, 0)) for _ in range(2)),
    out_shape=tuple(jax.ShapeDtypeStruct((NP, 16), f32) for _ in range(2)),
)


def _scale0_body(x_ref, dinvo_ref, y_ref):
    y_ref[...] = x_ref[...] * dinvo_ref[...][:, :1]


_scale0 = pl.pallas_call(
    _scale0_body,
    grid=(NP // STRIPE,),
    in_specs=[
        pl.BlockSpec((STRIPE, D), lambda i: (i, 0)),
        pl.BlockSpec((STRIPE, 16), lambda i: (i, 0)),
    ],
    out_specs=pl.BlockSpec((STRIPE, D), lambda i: (i, 0)),
    out_shape=jax.ShapeDtypeStruct((NP, D), f32),
)


def _update_body(z_ref, dinvi_ref, dinvo_ref, accp_ref, acc_ref, y_ref):
    x = z_ref[...] * dinvi_ref[...][:, :1]
    acc_ref[...] = accp_ref[...] + x
    y_ref[...] = x * dinvo_ref[...][:, :1]


_update = pl.pallas_call(
    _update_body,
    grid=(NP // STRIPE,),
    in_specs=[
        pl.BlockSpec((STRIPE, D), lambda i: (i, 0)),
        pl.BlockSpec((STRIPE, 16), lambda i: (i, 0)),
        pl.BlockSpec((STRIPE, 16), lambda i: (i, 0)),
        pl.BlockSpec((STRIPE, D), lambda i: (i, 0)),
    ],
    out_specs=tuple(pl.BlockSpec((STRIPE, D), lambda i: (i, 0)) for _ in range(2)),
    out_shape=tuple(jax.ShapeDtypeStruct((NP, D), f32) for _ in range(2)),
)


def _update_last_body(z_ref, dinvi_ref, accp_ref, acc_ref):
    acc_ref[...] = accp_ref[...] + z_ref[...] * dinvi_ref[...][:, :1]


_update_last = pl.pallas_call(
    _update_last_body,
    grid=(NP // STRIPE,),
    in_specs=[
        pl.BlockSpec((STRIPE, D), lambda i: (i, 0)),
        pl.BlockSpec((STRIPE, 16), lambda i: (i, 0)),
        pl.BlockSpec((STRIPE, D), lambda i: (i, 0)),
    ],
    out_specs=pl.BlockSpec((STRIPE, D), lambda i: (i, 0)),
    out_shape=jax.ShapeDtypeStruct((NP, D), f32),
)


BK = 512
NBK = 49


def _matmul_body(ue_ref, items_ref, out_ref):
    out_ref[...] = lax.dot_general(
        ue_ref[...], items_ref[...],
        (((1,), (1,)), ((), ())),
        preferred_element_type=f32,
    ) * (1.0 / 16.0)


_matmul = pl.pallas_call(
    _matmul_body,
    grid=(NBK,),
    in_specs=[
        pl.BlockSpec((B, D), lambda j: (0, 0)),
        pl.BlockSpec((BK, D), lambda j: (j + HALF // BK, 0)),
    ],
    out_specs=pl.BlockSpec((B, BK), lambda j: (0, j)),
    out_shape=jax.ShapeDtypeStruct((B, NU), f32),
)


def kernel(users, edge_index, user_embedding, item_embedding):
    src = edge_index[0].astype(i32)
    dst = edge_index[1].astype(i32)
    padn = EP - E
    fill = (lax.iota(i32, padn) * 131) % N
    srcp = jnp.concatenate([src, fill]).reshape(ROWS, GB)
    dstp = jnp.concatenate([dst, jnp.full((padn,), N, i32)]).reshape(ROWS, GB)

    srcg, dl0, dl1, sl0, sl1 = _prep(srcp, dstp)
    srcg3 = srcg.reshape(NT, GPT, GB)
    dloc4 = jnp.stack([dl0, dl1]).reshape(2, NT, GPT, GB)
    srcl4 = jnp.stack([sl0, sl1]).reshape(2, NT, GPT, GB)

    dego, degi = _deg(srcl4, dloc4)
    dinvo, dinvi = _dinv(dego, degi)

    zpad = jnp.zeros((HALF - NU, D), f32)
    x0 = jnp.concatenate([user_embedding, zpad, item_embedding, zpad], axis=0)
    acc = x0
    y = _scale0(x0, dinvo)
    for t in range(3):
        z = _spmm(y, srcg3, dloc4)
        if t < 2:
            acc, y = _update(z, dinvi, dinvo, acc)
        else:
            acc = _update_last(z, dinvi, acc)

    ue = _gather_users(acc, users.astype(i32))
    return _matmul(ue, acc)


# speedup vs baseline: 11.9398x; 1.2972x over previous
import functools

import jax
import jax.numpy as jnp
from jax import lax
from jax.experimental import pallas as pl
from jax.experimental.pallas import tpu as pltpu
from jax.experimental.pallas import tpu_sc as plsc

f32 = jnp.float32
i32 = jnp.int32

NU = 25000
N = 50000
D = 64
E = 800000
B = 1024
HALF = 25088
NP = 2 * HALF
NT = 16
GB = 128
GPT = 392
GRP = 8
NGRP = 49
EP = NT * GPT * GB
ROWS = NT * GPT
STRIPE = HALF // NT
NTRASH = 80

_mesh = plsc.VectorSubcoreMesh(core_axis_name="c", subcore_axis_name="s")


def _prep_body(src_ref, dst_ref, srcg_ref, dl0_ref, dl1_ref, sl0_ref, sl1_ref):
    src = src_ref[...]
    dst = dst_ref[...]
    srcg_ref[...] = src + 88 * (src >= NU).astype(i32)
    pid = pl.program_id(0)
    r = pid * GPT + lax.broadcasted_iota(i32, (GPT, GB), 0)
    l = lax.broadcasted_iota(i32, (GPT, GB), 1)
    toff = NU + ((r * GB + l) % NTRASH)
    dl0_ref[...] = jnp.where(dst < NU, dst, toff)
    dl1_ref[...] = jnp.where((dst >= NU) & (dst < N), dst - NU, toff)
    sl0_ref[...] = jnp.where(src < NU, src, toff)
    sl1_ref[...] = jnp.where((src >= NU) & (src < N), src - NU, toff)


_prep = pl.pallas_call(
    _prep_body,
    grid=(NT,),
    in_specs=[pl.BlockSpec((GPT, GB), lambda i: (i, 0)) for _ in range(2)],
    out_specs=tuple(pl.BlockSpec((GPT, GB), lambda i: (i, 0)) for _ in range(5)),
    out_shape=tuple(jax.ShapeDtypeStruct((ROWS, GB), i32) for _ in range(5)),
)


EPG = 224
NPAIR = 112
EPT = GPT * GB
DG = 1024
NDG = EPT // DG


@functools.partial(
    pl.kernel,
    out_type=(
        jax.ShapeDtypeStruct((NP, 16), f32),
        jax.ShapeDtypeStruct((NP, 16), f32),
    ),
    mesh=_mesh,
    compiler_params=pltpu.CompilerParams(use_tc_tiling_on_sc=False),
    scratch_types=[
        pltpu.VMEM((DG,), i32),
        pltpu.VMEM((DG,), i32),
        pltpu.VMEM((DG, 16), f32),
        pltpu.VMEM((STRIPE, 16), f32),
        pltpu.VMEM_SHARED((HALF, 16), f32),
        pltpu.VMEM_SHARED((HALF, 16), f32),
        pltpu.SemaphoreType.DMA,
    ],
)
def _deg(srcl_hbm, dloc_hbm, dego_hbm, degi_hbm, sidx, didx, ones_v, zv,
         dego_sh, degi_sh, semi):
    c = lax.axis_index("c")
    s = lax.axis_index("s")

    def fill(i, carry):
        ones_v[i, :] = jnp.ones((16,), f32)
        return carry

    lax.fori_loop(0, DG, fill, None)

    def fillz(i, carry):
        zv[i, :] = jnp.zeros((16,), f32)
        return carry

    lax.fori_loop(0, STRIPE, fillz, None)
    pltpu.sync_copy(zv, dego_sh.at[pl.ds(s * STRIPE, STRIPE)])
    pltpu.sync_copy(zv, degi_sh.at[pl.ds(s * STRIPE, STRIPE)])
    plsc.subcore_barrier()

    def group(h, carry):
        base = h * DG
        ca = pltpu.async_copy(srcl_hbm.at[c, s, pl.ds(base, DG)], sidx, semi)
        cb = pltpu.async_copy(dloc_hbm.at[c, s, pl.ds(base, DG)], didx, semi)
        ca.wait()
        cb.wait()
        pltpu.sync_copy(ones_v, dego_sh.at[sidx], add=True)
        pltpu.sync_copy(ones_v, degi_sh.at[didx], add=True)
        return carry

    lax.fori_loop(0, NDG, group, None)
    plsc.subcore_barrier()
    dst_lo = c * HALF + s * STRIPE
    pltpu.sync_copy(dego_sh.at[pl.ds(s * STRIPE, STRIPE)],
                    dego_hbm.at[pl.ds(dst_lo, STRIPE)])
    pltpu.sync_copy(degi_sh.at[pl.ds(s * STRIPE, STRIPE)],
                    degi_hbm.at[pl.ds(dst_lo, STRIPE)])


@functools.partial(
    pl.kernel,
    out_type=jax.ShapeDtypeStruct((NP, D), f32),
    mesh=_mesh,
    compiler_params=pltpu.CompilerParams(use_tc_tiling_on_sc=False),
    scratch_types=[
        pltpu.VMEM((EPG,), i32),
        pltpu.VMEM((EPG,), i32),
        pltpu.VMEM((EPG,), i32),
        pltpu.VMEM((EPG,), i32),
        pltpu.VMEM((EPG, D), f32),
        pltpu.VMEM((EPG, D), f32),
        pltpu.VMEM_SHARED((HALF, D), f32),
        pltpu.SemaphoreType.DMA,
        pltpu.SemaphoreType.DMA,
        pltpu.SemaphoreType.DMA,
    ],
)
def _spmm(y_hbm, srcg_hbm, dloc_hbm, z_hbm, sidxa, sidxb, didxa, didxb,
          rowsa, rowsb, acc_sh, sema, semb, semi):
    c = lax.axis_index("c")
    s = lax.axis_index("s")

    def fillz(i, carry):
        for k in range(4):
            rowsa[i, pl.ds(k * 16, 16)] = jnp.zeros((16,), f32)
        return carry

    lax.fori_loop(0, EPG, fillz, None)
    for q in range(7):
        pltpu.sync_copy(rowsa, acc_sh.at[pl.ds(s * STRIPE + q * EPG, EPG)])
    plsc.subcore_barrier()

    def group(h, carry):
        base = h * 2 * EPG
        i1 = pltpu.async_copy(srcg_hbm.at[s, pl.ds(base, EPG)], sidxa, semi)
        i2 = pltpu.async_copy(srcg_hbm.at[s, pl.ds(base + EPG, EPG)], sidxb,
                              semi)
        i3 = pltpu.async_copy(dloc_hbm.at[c, s, pl.ds(base, EPG)], didxa,
                              semi)
        i4 = pltpu.async_copy(dloc_hbm.at[c, s, pl.ds(base + EPG, EPG)],
                              didxb, semi)
        i1.wait()
        ga = pltpu.async_copy(y_hbm.at[sidxa], rowsa, sema)
        i2.wait()
        gb = pltpu.async_copy(y_hbm.at[sidxb], rowsb, semb)
        i3.wait()
        i4.wait()
        ga.wait()
        pltpu.sync_copy(rowsa, acc_sh.at[didxa], add=True)
        gb.wait()
        pltpu.sync_copy(rowsb, acc_sh.at[didxb], add=True)
        return carry

    lax.fori_loop(0, NPAIR, group, None)
    plsc.subcore_barrier()
    pltpu.sync_copy(acc_sh.at[pl.ds(s * STRIPE, STRIPE)],
                    z_hbm.at[pl.ds(c * HALF + s * STRIPE, STRIPE)])


@functools.partial(
    pl.kernel,
    out_type=jax.ShapeDtypeStruct((B, D), f32),
    mesh=_mesh,
    compiler_params=pltpu.CompilerParams(use_tc_tiling_on_sc=False),
    scratch_types=[
        pltpu.VMEM((B // 32,), i32),
        pltpu.VMEM((B // 32, D), f32),
        pltpu.SemaphoreType.DMA,
    ],
)
def _gather_users(acc_hbm, users_hbm, ue_hbm, uidx, urows, sem):
    wid = lax.axis_index("s") * 2 + lax.axis_index("c")
    base = wid * (B // 32)
    pltpu.sync_copy(users_hbm.at[pl.ds(base, B // 32)], uidx)
    pltpu.async_copy(acc_hbm.at[uidx], urows, sem).wait()
    pltpu.sync_copy(urows, ue_hbm.at[pl.ds(base, B // 32)])


def _dinv_body(dego_ref, degi_ref, dinvo_ref, dinvi_ref):
    dinvo_ref[...] = lax.rsqrt(jnp.maximum(dego_ref[...], 1.0))
    dinvi_ref[...] = lax.rsqrt(jnp.maximum(degi_ref[...], 1.0))


_dinv = pl.pallas_call(
    _dinv_body,
    grid=(NP // STRIPE,),
    in_specs=[pl.BlockSpec((STRIPE, 16), lambda i: (i, 0)) for _ in range(2)],
    out_specs=tuple(pl.BlockSpec((STRIPE, 16), lambda i: (i, 0)) for _ in range(2)),
    out_shape=tuple(jax.ShapeDtypeStruct((NP, 16), f32) for _ in range(2)),
)


def _scale0_body(x_ref, dinvo_ref, y_ref):
    y_ref[...] = x_ref[...] * dinvo_ref[...][:, :1]


_scale0 = pl.pallas_call(
    _scale0_body,
    grid=(NP // STRIPE,),
    in_specs=[
        pl.BlockSpec((STRIPE, D), lambda i: (i, 0)),
        pl.BlockSpec((STRIPE, 16), lambda i: (i, 0)),
    ],
    out_specs=pl.BlockSpec((STRIPE, D), lambda i: (i, 0)),
    out_shape=jax.ShapeDtypeStruct((NP, D), f32),
)


def _update_body(z_ref, dinvi_ref, dinvo_ref, accp_ref, acc_ref, y_ref):
    x = z_ref[...] * dinvi_ref[...][:, :1]
    acc_ref[...] = accp_ref[...] + x
    y_ref[...] = x * dinvo_ref[...][:, :1]


_update = pl.pallas_call(
    _update_body,
    grid=(NP // STRIPE,),
    in_specs=[
        pl.BlockSpec((STRIPE, D), lambda i: (i, 0)),
        pl.BlockSpec((STRIPE, 16), lambda i: (i, 0)),
        pl.BlockSpec((STRIPE, 16), lambda i: (i, 0)),
        pl.BlockSpec((STRIPE, D), lambda i: (i, 0)),
    ],
    out_specs=tuple(pl.BlockSpec((STRIPE, D), lambda i: (i, 0)) for _ in range(2)),
    out_shape=tuple(jax.ShapeDtypeStruct((NP, D), f32) for _ in range(2)),
)


def _update_last_body(z_ref, dinvi_ref, accp_ref, acc_ref):
    acc_ref[...] = accp_ref[...] + z_ref[...] * dinvi_ref[...][:, :1]


_update_last = pl.pallas_call(
    _update_last_body,
    grid=(NP // STRIPE,),
    in_specs=[
        pl.BlockSpec((STRIPE, D), lambda i: (i, 0)),
        pl.BlockSpec((STRIPE, 16), lambda i: (i, 0)),
        pl.BlockSpec((STRIPE, D), lambda i: (i, 0)),
    ],
    out_specs=pl.BlockSpec((STRIPE, D), lambda i: (i, 0)),
    out_shape=jax.ShapeDtypeStruct((NP, D), f32),
)


BK = 512
NBK = 49


def _matmul_body(ue_ref, items_ref, out_ref):
    out_ref[...] = lax.dot_general(
        ue_ref[...], items_ref[...],
        (((1,), (1,)), ((), ())),
        preferred_element_type=f32,
    ) * (1.0 / 16.0)


_matmul = pl.pallas_call(
    _matmul_body,
    grid=(NBK,),
    in_specs=[
        pl.BlockSpec((B, D), lambda j: (0, 0)),
        pl.BlockSpec((BK, D), lambda j: (j + HALF // BK, 0)),
    ],
    out_specs=pl.BlockSpec((B, BK), lambda j: (0, j)),
    out_shape=jax.ShapeDtypeStruct((B, NU), f32),
)


def kernel(users, edge_index, user_embedding, item_embedding):
    src = edge_index[0].astype(i32)
    dst = edge_index[1].astype(i32)
    padn = EP - E
    fill = (lax.iota(i32, padn) * 131) % N
    srcp = jnp.concatenate([src, fill]).reshape(ROWS, GB)
    dstp = jnp.concatenate([dst, jnp.full((padn,), N, i32)]).reshape(ROWS, GB)

    srcg, dl0, dl1, sl0, sl1 = _prep(srcp, dstp)
    srcg3 = srcg.reshape(NT, EPT)
    dloc4 = jnp.stack([dl0, dl1]).reshape(2, NT, EPT)
    srcl4 = jnp.stack([sl0, sl1]).reshape(2, NT, EPT)

    dego, degi = _deg(srcl4, dloc4)
    dinvo, dinvi = _dinv(dego, degi)

    zpad = jnp.zeros((HALF - NU, D), f32)
    x0 = jnp.concatenate([user_embedding, zpad, item_embedding, zpad], axis=0)
    acc = x0
    y = _scale0(x0, dinvo)
    for t in range(3):
        z = _spmm(y, srcg3, dloc4)
        if t < 2:
            acc, y = _update(z, dinvi, dinvo, acc)
        else:
            acc = _update_last(z, dinvi, acc)

    ue = _gather_users(acc, users.astype(i32))
    return _matmul(ue, acc)


# speedup vs baseline: 15.7767x; 1.3213x over previous
import functools

import jax
import jax.numpy as jnp
from jax import lax
from jax.experimental import pallas as pl
from jax.experimental.pallas import tpu as pltpu
from jax.experimental.pallas import tpu_sc as plsc

f32 = jnp.float32
i32 = jnp.int32

NU = 25000
N = 50000
D = 64
E = 800000
B = 1024
HALF = 25088
NP = 2 * HALF
NT = 16
GB = 128
GPT = 392
GRP = 8
NGRP = 49
EP = NT * GPT * GB
ROWS = NT * GPT
STRIPE = HALF // NT
NTRASH = 80

_mesh = plsc.VectorSubcoreMesh(core_axis_name="c", subcore_axis_name="s")


def _prep_body(src_ref, dst_ref, srcg_ref, dl0_ref, dl1_ref, sl0_ref, sl1_ref):
    src = src_ref[...]
    dst = dst_ref[...]
    srcg_ref[...] = src + 88 * (src >= NU).astype(i32)
    pid = pl.program_id(0)
    r = pid * GPT + lax.broadcasted_iota(i32, (GPT, GB), 0)
    l = lax.broadcasted_iota(i32, (GPT, GB), 1)
    toff = NU + ((r * GB + l) % NTRASH)
    dl0_ref[...] = jnp.where(dst < NU, dst, toff)
    dl1_ref[...] = jnp.where((dst >= NU) & (dst < N), dst - NU, toff)
    sl0_ref[...] = jnp.where(src < NU, src, toff)
    sl1_ref[...] = jnp.where((src >= NU) & (src < N), src - NU, toff)


_prep = pl.pallas_call(
    _prep_body,
    grid=(NT,),
    in_specs=[pl.BlockSpec((GPT, GB), lambda i: (i, 0)) for _ in range(2)],
    out_specs=tuple(pl.BlockSpec((GPT, GB), lambda i: (i, 0)) for _ in range(5)),
    out_shape=tuple(jax.ShapeDtypeStruct((ROWS, GB), i32) for _ in range(5)),
)


EPG = 224
NPAIR = 112
EPT = GPT * GB
DG = 1024
NDG = EPT // DG


@functools.partial(
    pl.kernel,
    out_type=(
        jax.ShapeDtypeStruct((NP, 16), f32),
        jax.ShapeDtypeStruct((NP, 16), f32),
    ),
    mesh=_mesh,
    compiler_params=pltpu.CompilerParams(use_tc_tiling_on_sc=False),
    scratch_types=[
        pltpu.VMEM((DG,), i32),
        pltpu.VMEM((DG,), i32),
        pltpu.VMEM((DG, 16), f32),
        pltpu.VMEM((STRIPE, 16), f32),
        pltpu.VMEM_SHARED((HALF, 16), f32),
        pltpu.VMEM_SHARED((HALF, 16), f32),
        pltpu.SemaphoreType.DMA,
    ],
)
def _deg(srcl_hbm, dloc_hbm, dego_hbm, degi_hbm, sidx, didx, ones_v, zv,
         dego_sh, degi_sh, semi):
    c = lax.axis_index("c")
    s = lax.axis_index("s")

    def fill(i, carry):
        ones_v[i, :] = jnp.ones((16,), f32)
        return carry

    lax.fori_loop(0, DG, fill, None)

    def fillz(i, carry):
        zv[i, :] = jnp.zeros((16,), f32)
        return carry

    lax.fori_loop(0, STRIPE, fillz, None)
    pltpu.sync_copy(zv, dego_sh.at[pl.ds(s * STRIPE, STRIPE)])
    pltpu.sync_copy(zv, degi_sh.at[pl.ds(s * STRIPE, STRIPE)])
    plsc.subcore_barrier()

    def group(h, carry):
        base = h * DG
        ca = pltpu.async_copy(srcl_hbm.at[c, s, pl.ds(base, DG)], sidx, semi)
        cb = pltpu.async_copy(dloc_hbm.at[c, s, pl.ds(base, DG)], didx, semi)
        ca.wait()
        cb.wait()
        pltpu.sync_copy(ones_v, dego_sh.at[sidx], add=True)
        pltpu.sync_copy(ones_v, degi_sh.at[didx], add=True)
        return carry

    lax.fori_loop(0, NDG, group, None)
    plsc.subcore_barrier()
    dst_lo = c * HALF + s * STRIPE
    pltpu.sync_copy(dego_sh.at[pl.ds(s * STRIPE, STRIPE)],
                    dego_hbm.at[pl.ds(dst_lo, STRIPE)])
    pltpu.sync_copy(degi_sh.at[pl.ds(s * STRIPE, STRIPE)],
                    degi_hbm.at[pl.ds(dst_lo, STRIPE)])


EPT2 = EP // 32
CAP = EPT2 + 2 * EPG
PSTG = 6272


@functools.partial(
    pl.kernel,
    out_type=(
        jax.ShapeDtypeStruct((2, 32, CAP), i32),
        jax.ShapeDtypeStruct((2, 32, CAP), i32),
        jax.ShapeDtypeStruct((2, 32, 16), i32),
    ),
    mesh=_mesh,
    compiler_params=pltpu.CompilerParams(use_tc_tiling_on_sc=False,
                                         needs_layout_passes=False),
    scratch_types=[
        pltpu.VMEM((PSTG,), i32),
        pltpu.VMEM((PSTG,), i32),
        pltpu.VMEM((CAP,), i32),
        pltpu.VMEM((CAP,), i32),
        pltpu.VMEM((CAP,), i32),
        pltpu.VMEM((CAP,), i32),
        pltpu.VMEM((16,), i32),
    ],
)
def _part(srcp_hbm, dstp_hbm, psrc_hbm, pdst_hbm, pcnt_hbm,
          sbuf, dbuf, sl0, dl0, sl1, dl1, cntv):
    c = lax.axis_index("c")
    s = lax.axis_index("s")
    w = c * NT + s
    lane = lax.iota(i32, 16)

    def outer(g, carry):
        pltpu.sync_copy(srcp_hbm.at[w, pl.ds(g * PSTG, PSTG)], sbuf)
        pltpu.sync_copy(dstp_hbm.at[w, pl.ds(g * PSTG, PSTG)], dbuf)

        def inner(i, carry2):
            c0, c1 = carry2
            sv = sbuf[pl.ds(i * 16, 16)]
            dv = dbuf[pl.ds(i * 16, 16)]
            sg = sv + 88 * (sv >= NU).astype(i32)
            m0 = dv < NU
            m1i = (dv >= NU) & (dv < N)
            m0i = m0.astype(i32)
            m1 = m1i.astype(i32)
            pos0 = c0 + plsc.cumsum(m0i) - m0i
            pos1 = c1 + plsc.cumsum(m1) - m1
            idx0 = jnp.where(m0, pos0, CAP - 16 + lane)
            idx1 = jnp.where(m1i, pos1, CAP - 16 + lane)
            plsc.store_scatter(sl0, [idx0], sg)
            plsc.store_scatter(dl0, [idx0], dv)
            plsc.store_scatter(sl1, [idx1], sg)
            plsc.store_scatter(dl1, [idx1], dv - NU)
            return (c0 + plsc.all_reduce_population_count(m0),
                    c1 + plsc.all_reduce_population_count(m1i))

        return lax.fori_loop(0, PSTG // 16, inner, carry)

    zv16 = jnp.zeros((16,), i32)
    cnt0, cnt1 = lax.fori_loop(0, EPT2 // PSTG, outer, (zv16, zv16))

    trash = NU + lane
    for k in range(2 * EPG // 16):
        off = k * 16 + lane
        plsc.store_scatter(sl0, [cnt0 + off], lane)
        plsc.store_scatter(dl0, [cnt0 + off], trash)
        plsc.store_scatter(sl1, [cnt1 + off], lane)
        plsc.store_scatter(dl1, [cnt1 + off], trash)
    pltpu.sync_copy(sl0, psrc_hbm.at[0, w])
    pltpu.sync_copy(dl0, pdst_hbm.at[0, w])
    pltpu.sync_copy(sl1, psrc_hbm.at[1, w])
    pltpu.sync_copy(dl1, pdst_hbm.at[1, w])
    cntv[pl.ds(0, 16)] = cnt0
    pltpu.sync_copy(cntv, pcnt_hbm.at[0, w])
    cntv[pl.ds(0, 16)] = cnt1
    pltpu.sync_copy(cntv, pcnt_hbm.at[1, w])


@functools.partial(
    pl.kernel,
    out_type=jax.ShapeDtypeStruct((NP, D), f32),
    mesh=_mesh,
    compiler_params=pltpu.CompilerParams(use_tc_tiling_on_sc=False),
    scratch_types=[
        pltpu.VMEM((EPG,), i32),
        pltpu.VMEM((EPG,), i32),
        pltpu.VMEM((EPG,), i32),
        pltpu.VMEM((EPG,), i32),
        pltpu.VMEM((2, 16), i32),
        pltpu.VMEM((EPG, D), f32),
        pltpu.VMEM((EPG, D), f32),
        pltpu.VMEM_SHARED((HALF, D), f32),
        pltpu.SemaphoreType.DMA,
        pltpu.SemaphoreType.DMA,
        pltpu.SemaphoreType.DMA,
    ],
)
def _spmm(y_hbm, psrc_hbm, pdst_hbm, pcnt_hbm, z_hbm, sidxa, sidxb, didxa,
          didxb, cbuf, rowsa, rowsb, acc_sh, sema, semb, semi):
    c = lax.axis_index("c")
    s = lax.axis_index("s")

    def fillz(i, carry):
        for k in range(4):
            rowsa[i, pl.ds(k * 16, 16)] = jnp.zeros((16,), f32)
        return carry

    lax.fori_loop(0, EPG, fillz, None)
    for q in range(7):
        pltpu.sync_copy(rowsa, acc_sh.at[pl.ds(s * STRIPE + q * EPG, EPG)])
    plsc.subcore_barrier()

    pltpu.sync_copy(pcnt_hbm.at[c, pl.ds(2 * s, 2)], cbuf)

    def group_for(p):
        def group(h, carry):
            base = h * 2 * EPG
            i1 = pltpu.async_copy(psrc_hbm.at[c, p, pl.ds(base, EPG)],
                                  sidxa, semi)
            i2 = pltpu.async_copy(psrc_hbm.at[c, p, pl.ds(base + EPG, EPG)],
                                  sidxb, semi)
            i3 = pltpu.async_copy(pdst_hbm.at[c, p, pl.ds(base, EPG)],
                                  didxa, semi)
            i4 = pltpu.async_copy(pdst_hbm.at[c, p, pl.ds(base + EPG, EPG)],
                                  didxb, semi)
            i1.wait()
            ga = pltpu.async_copy(y_hbm.at[sidxa], rowsa, sema)
            i2.wait()
            gb = pltpu.async_copy(y_hbm.at[sidxb], rowsb, semb)
            i3.wait()
            i4.wait()
            ga.wait()
            pltpu.sync_copy(rowsa, acc_sh.at[didxa], add=True)
            gb.wait()
            pltpu.sync_copy(rowsb, acc_sh.at[didxb], add=True)
            return carry

        return group

    for pi in range(2):
        cv = cbuf[pi, :]
        n = cv[0]
        nblk = (n + 2 * EPG - 1) // (2 * EPG)
        lax.fori_loop(0, nblk, group_for(2 * s + pi), None)

    plsc.subcore_barrier()
    pltpu.sync_copy(acc_sh.at[pl.ds(s * STRIPE, STRIPE)],
                    z_hbm.at[pl.ds(c * HALF + s * STRIPE, STRIPE)])


@functools.partial(
    pl.kernel,
    out_type=jax.ShapeDtypeStruct((B, D), f32),
    mesh=_mesh,
    compiler_params=pltpu.CompilerParams(use_tc_tiling_on_sc=False),
    scratch_types=[
        pltpu.VMEM((B // 32,), i32),
        pltpu.VMEM((B // 32, D), f32),
        pltpu.SemaphoreType.DMA,
    ],
)
def _gather_users(acc_hbm, users_hbm, ue_hbm, uidx, urows, sem):
    wid = lax.axis_index("s") * 2 + lax.axis_index("c")
    base = wid * (B // 32)
    pltpu.sync_copy(users_hbm.at[pl.ds(base, B // 32)], uidx)
    pltpu.async_copy(acc_hbm.at[uidx], urows, sem).wait()
    pltpu.sync_copy(urows, ue_hbm.at[pl.ds(base, B // 32)])


def _dinv_body(dego_ref, degi_ref, dinvo_ref, dinvi_ref):
    dinvo_ref[...] = lax.rsqrt(jnp.maximum(dego_ref[...], 1.0))
    dinvi_ref[...] = lax.rsqrt(jnp.maximum(degi_ref[...], 1.0))


_dinv = pl.pallas_call(
    _dinv_body,
    grid=(NP // STRIPE,),
    in_specs=[pl.BlockSpec((STRIPE, 16), lambda i: (i, 0)) for _ in range(2)],
    out_specs=tuple(pl.BlockSpec((STRIPE, 16), lambda i: (i, 0)) for _ in range(2)),
    out_shape=tuple(jax.ShapeDtypeStruct((NP, 16), f32) for _ in range(2)),
)


def _scale0_body(x_ref, dinvo_ref, y_ref):
    y_ref[...] = x_ref[...] * dinvo_ref[...][:, :1]


_scale0 = pl.pallas_call(
    _scale0_body,
    grid=(NP // STRIPE,),
    in_specs=[
        pl.BlockSpec((STRIPE, D), lambda i: (i, 0)),
        pl.BlockSpec((STRIPE, 16), lambda i: (i, 0)),
    ],
    out_specs=pl.BlockSpec((STRIPE, D), lambda i: (i, 0)),
    out_shape=jax.ShapeDtypeStruct((NP, D), f32),
)


def _update_body(z_ref, dinvi_ref, dinvo_ref, accp_ref, acc_ref, y_ref):
    x = z_ref[...] * dinvi_ref[...][:, :1]
    acc_ref[...] = accp_ref[...] + x
    y_ref[...] = x * dinvo_ref[...][:, :1]


_update = pl.pallas_call(
    _update_body,
    grid=(NP // STRIPE,),
    in_specs=[
        pl.BlockSpec((STRIPE, D), lambda i: (i, 0)),
        pl.BlockSpec((STRIPE, 16), lambda i: (i, 0)),
        pl.BlockSpec((STRIPE, 16), lambda i: (i, 0)),
        pl.BlockSpec((STRIPE, D), lambda i: (i, 0)),
    ],
    out_specs=tuple(pl.BlockSpec((STRIPE, D), lambda i: (i, 0)) for _ in range(2)),
    out_shape=tuple(jax.ShapeDtypeStruct((NP, D), f32) for _ in range(2)),
)


def _update_last_body(z_ref, dinvi_ref, accp_ref, acc_ref):
    acc_ref[...] = accp_ref[...] + z_ref[...] * dinvi_ref[...][:, :1]


_update_last = pl.pallas_call(
    _update_last_body,
    grid=(NP // STRIPE,),
    in_specs=[
        pl.BlockSpec((STRIPE, D), lambda i: (i, 0)),
        pl.BlockSpec((STRIPE, 16), lambda i: (i, 0)),
        pl.BlockSpec((STRIPE, D), lambda i: (i, 0)),
    ],
    out_specs=pl.BlockSpec((STRIPE, D), lambda i: (i, 0)),
    out_shape=jax.ShapeDtypeStruct((NP, D), f32),
)


BK = 512
NBK = 49


def _matmul_body(ue_ref, items_ref, out_ref):
    out_ref[...] = lax.dot_general(
        ue_ref[...], items_ref[...],
        (((1,), (1,)), ((), ())),
        preferred_element_type=f32,
    ) * (1.0 / 16.0)


_matmul = pl.pallas_call(
    _matmul_body,
    grid=(NBK,),
    in_specs=[
        pl.BlockSpec((B, D), lambda j: (0, 0)),
        pl.BlockSpec((BK, D), lambda j: (j + HALF // BK, 0)),
    ],
    out_specs=pl.BlockSpec((B, BK), lambda j: (0, j)),
    out_shape=jax.ShapeDtypeStruct((B, NU), f32),
)


def kernel(users, edge_index, user_embedding, item_embedding):
    src = edge_index[0].astype(i32)
    dst = edge_index[1].astype(i32)
    padn = EP - E
    fill = (lax.iota(i32, padn) * 131) % N
    srcp = jnp.concatenate([src, fill]).reshape(ROWS, GB)
    dstp = jnp.concatenate([dst, jnp.full((padn,), N, i32)]).reshape(ROWS, GB)

    srcg, dl0, dl1, sl0, sl1 = _prep(srcp, dstp)
    dloc4 = jnp.stack([dl0, dl1]).reshape(2, NT, EPT)
    srcl4 = jnp.stack([sl0, sl1]).reshape(2, NT, EPT)

    psrc, pdst, pcnt = _part(srcp.reshape(32, EPT2), dstp.reshape(32, EPT2))
    dego, degi = _deg(srcl4, dloc4)
    dinvo, dinvi = _dinv(dego, degi)

    zpad = jnp.zeros((HALF - NU, D), f32)
    x0 = jnp.concatenate([user_embedding, zpad, item_embedding, zpad], axis=0)
    acc = x0
    y = _scale0(x0, dinvo)
    for t in range(3):
        z = _spmm(y, psrc, pdst, pcnt)
        if t < 2:
            acc, y = _update(z, dinvi, dinvo, acc)
        else:
            acc = _update_last(z, dinvi, acc)

    ue = _gather_users(acc, users.astype(i32))
    return _matmul(ue, acc)


# speedup vs baseline: 16.4460x; 1.0424x over previous
import functools

import jax
import jax.numpy as jnp
from jax import lax
from jax.experimental import pallas as pl
from jax.experimental.pallas import tpu as pltpu
from jax.experimental.pallas import tpu_sc as plsc

f32 = jnp.float32
i32 = jnp.int32

NU = 25000
N = 50000
D = 64
E = 800000
B = 1024
HALF = 25088
NP = 2 * HALF
NT = 16
GB = 128
GPT = 392
GRP = 8
NGRP = 49
EP = NT * GPT * GB
ROWS = NT * GPT
STRIPE = HALF // NT
NTRASH = 80

_mesh = plsc.VectorSubcoreMesh(core_axis_name="c", subcore_axis_name="s")


def _prep_body(src_ref, dst_ref, dl0_ref, dl1_ref, sl0_ref, sl1_ref):
    src = src_ref[...]
    dst = dst_ref[...]
    pid = pl.program_id(0)
    r = pid * GPT + lax.broadcasted_iota(i32, (GPT, GB), 0)
    l = lax.broadcasted_iota(i32, (GPT, GB), 1)
    toff = NU + ((r * GB + l) % NTRASH)
    dl0_ref[...] = jnp.where(dst < NU, dst, toff)
    dl1_ref[...] = jnp.where((dst >= NU) & (dst < N), dst - NU, toff)
    sl0_ref[...] = jnp.where(src < NU, src, toff)
    sl1_ref[...] = jnp.where((src >= NU) & (src < N), src - NU, toff)


_prep = pl.pallas_call(
    _prep_body,
    grid=(NT,),
    in_specs=[pl.BlockSpec((GPT, GB), lambda i: (i, 0)) for _ in range(2)],
    out_specs=tuple(pl.BlockSpec((GPT, GB), lambda i: (i, 0)) for _ in range(4)),
    out_shape=tuple(jax.ShapeDtypeStruct((ROWS, GB), i32) for _ in range(4)),
)


EPG = 224
NPAIR = 112
EPT = GPT * GB
DG = 1024
NDG = EPT // DG


@functools.partial(
    pl.kernel,
    out_type=(
        jax.ShapeDtypeStruct((NP, 16), f32),
        jax.ShapeDtypeStruct((NP, 16), f32),
    ),
    mesh=_mesh,
    compiler_params=pltpu.CompilerParams(use_tc_tiling_on_sc=False),
    scratch_types=[
        pltpu.VMEM((DG,), i32),
        pltpu.VMEM((DG,), i32),
        pltpu.VMEM((DG, 16), f32),
        pltpu.VMEM((STRIPE, 16), f32),
        pltpu.VMEM_SHARED((HALF, 16), f32),
        pltpu.VMEM_SHARED((HALF, 16), f32),
        pltpu.SemaphoreType.DMA,
    ],
)
def _deg(srcl_hbm, dloc_hbm, dego_hbm, degi_hbm, sidx, didx, ones_v, zv,
         dego_sh, degi_sh, semi):
    c = lax.axis_index("c")
    s = lax.axis_index("s")

    def fill(i, carry):
        ones_v[i, :] = jnp.ones((16,), f32)
        return carry

    lax.fori_loop(0, DG, fill, None)

    def fillz(i, carry):
        zv[i, :] = jnp.zeros((16,), f32)
        return carry

    lax.fori_loop(0, STRIPE, fillz, None)
    pltpu.sync_copy(zv, dego_sh.at[pl.ds(s * STRIPE, STRIPE)])
    pltpu.sync_copy(zv, degi_sh.at[pl.ds(s * STRIPE, STRIPE)])
    plsc.subcore_barrier()

    def group(h, carry):
        base = h * DG
        ca = pltpu.async_copy(srcl_hbm.at[c, s, pl.ds(base, DG)], sidx, semi)
        cb = pltpu.async_copy(dloc_hbm.at[c, s, pl.ds(base, DG)], didx, semi)
        ca.wait()
        cb.wait()
        pltpu.sync_copy(ones_v, dego_sh.at[sidx], add=True)
        pltpu.sync_copy(ones_v, degi_sh.at[didx], add=True)
        return carry

    lax.fori_loop(0, NDG, group, None)
    plsc.subcore_barrier()
    dst_lo = c * HALF + s * STRIPE
    pltpu.sync_copy(dego_sh.at[pl.ds(s * STRIPE, STRIPE)],
                    dego_hbm.at[pl.ds(dst_lo, STRIPE)])
    pltpu.sync_copy(degi_sh.at[pl.ds(s * STRIPE, STRIPE)],
                    degi_hbm.at[pl.ds(dst_lo, STRIPE)])


EPT2 = EP // 32
CAP = EPT2 + 2 * EPG
PSTG = 6272


@functools.partial(
    pl.kernel,
    out_type=(
        jax.ShapeDtypeStruct((2, 32, CAP), i32),
        jax.ShapeDtypeStruct((2, 32, CAP), i32),
        jax.ShapeDtypeStruct((2, 32, 16), i32),
    ),
    mesh=_mesh,
    compiler_params=pltpu.CompilerParams(use_tc_tiling_on_sc=False,
                                         needs_layout_passes=False),
    scratch_types=[
        pltpu.VMEM((PSTG,), i32),
        pltpu.VMEM((PSTG,), i32),
        pltpu.VMEM((CAP,), i32),
        pltpu.VMEM((CAP,), i32),
        pltpu.VMEM((CAP,), i32),
        pltpu.VMEM((CAP,), i32),
        pltpu.VMEM((16,), i32),
    ],
)
def _part(srcp_hbm, dstp_hbm, psrc_hbm, pdst_hbm, pcnt_hbm,
          sbuf, dbuf, sl0, dl0, sl1, dl1, cntv):
    c = lax.axis_index("c")
    s = lax.axis_index("s")
    w = c * NT + s
    lane = lax.iota(i32, 16)

    def outer(g, carry):
        pltpu.sync_copy(srcp_hbm.at[w, pl.ds(g * PSTG, PSTG)], sbuf)
        pltpu.sync_copy(dstp_hbm.at[w, pl.ds(g * PSTG, PSTG)], dbuf)

        def inner(i, carry2):
            c0, c1 = carry2
            sv = sbuf[pl.ds(i * 16, 16)]
            dv = dbuf[pl.ds(i * 16, 16)]
            sg = sv + 88 * (sv >= NU).astype(i32)
            m0 = dv < NU
            m1i = (dv >= NU) & (dv < N)
            m0i = m0.astype(i32)
            m1 = m1i.astype(i32)
            pos0 = c0 + plsc.cumsum(m0i) - m0i
            pos1 = c1 + plsc.cumsum(m1) - m1
            idx0 = jnp.where(m0, pos0, CAP - 16 + lane)
            idx1 = jnp.where(m1i, pos1, CAP - 16 + lane)
            plsc.store_scatter(sl0, [idx0], sg)
            plsc.store_scatter(dl0, [idx0], dv)
            plsc.store_scatter(sl1, [idx1], sg)
            plsc.store_scatter(dl1, [idx1], dv - NU)
            return (c0 + plsc.all_reduce_population_count(m0),
                    c1 + plsc.all_reduce_population_count(m1i))

        return lax.fori_loop(0, PSTG // 16, inner, carry)

    zv16 = jnp.zeros((16,), i32)
    cnt0, cnt1 = lax.fori_loop(0, EPT2 // PSTG, outer, (zv16, zv16))

    trash = NU + lane
    for k in range(2 * EPG // 16):
        off = k * 16 + lane
        plsc.store_scatter(sl0, [cnt0 + off], lane)
        plsc.store_scatter(dl0, [cnt0 + off], trash)
        plsc.store_scatter(sl1, [cnt1 + off], lane)
        plsc.store_scatter(dl1, [cnt1 + off], trash)
    pltpu.sync_copy(sl0, psrc_hbm.at[0, w])
    pltpu.sync_copy(dl0, pdst_hbm.at[0, w])
    pltpu.sync_copy(sl1, psrc_hbm.at[1, w])
    pltpu.sync_copy(dl1, pdst_hbm.at[1, w])
    cntv[pl.ds(0, 16)] = cnt0
    pltpu.sync_copy(cntv, pcnt_hbm.at[0, w])
    cntv[pl.ds(0, 16)] = cnt1
    pltpu.sync_copy(cntv, pcnt_hbm.at[1, w])


@functools.partial(
    pl.kernel,
    out_type=jax.ShapeDtypeStruct((NP, D), f32),
    mesh=_mesh,
    compiler_params=pltpu.CompilerParams(use_tc_tiling_on_sc=False),
    scratch_types=[
        pltpu.VMEM((EPG,), i32),
        pltpu.VMEM((EPG,), i32),
        pltpu.VMEM((EPG,), i32),
        pltpu.VMEM((EPG,), i32),
        pltpu.VMEM((2, 16), i32),
        pltpu.VMEM((EPG,), f32),
        pltpu.VMEM((EPG, D), f32),
        pltpu.VMEM((EPG, D), f32),
        pltpu.VMEM_SHARED((HALF, D), f32),
        pltpu.SemaphoreType.DMA,
        pltpu.SemaphoreType.DMA,
        pltpu.SemaphoreType.DMA,
    ],
)
def _spmm(y_hbm, psrc_hbm, pdst_hbm, pcnt_hbm, w2_hbm, z_hbm, sidxa, sidxb,
          didxa, didxb, cbuf, wbuf, rowsa, rowsb, acc_sh, sema, semb, semi):
    c = lax.axis_index("c")
    s = lax.axis_index("s")

    def fillz(i, carry):
        for k in range(4):
            rowsa[i, pl.ds(k * 16, 16)] = jnp.zeros((16,), f32)
        return carry

    lax.fori_loop(0, EPG, fillz, None)
    for q in range(7):
        pltpu.sync_copy(rowsa, acc_sh.at[pl.ds(s * STRIPE + q * EPG, EPG)])
    plsc.subcore_barrier()

    pltpu.sync_copy(pcnt_hbm.at[c, pl.ds(2 * s, 2)], cbuf)

    def group_for(p):
        def group(h, carry):
            base = h * 2 * EPG
            i1 = pltpu.async_copy(psrc_hbm.at[c, p, pl.ds(base, EPG)],
                                  sidxa, semi)
            i2 = pltpu.async_copy(psrc_hbm.at[c, p, pl.ds(base + EPG, EPG)],
                                  sidxb, semi)
            i3 = pltpu.async_copy(pdst_hbm.at[c, p, pl.ds(base, EPG)],
                                  didxa, semi)
            i4 = pltpu.async_copy(pdst_hbm.at[c, p, pl.ds(base + EPG, EPG)],
                                  didxb, semi)
            i1.wait()
            ga = pltpu.async_copy(y_hbm.at[sidxa], rowsa, sema)
            i2.wait()
            gb = pltpu.async_copy(y_hbm.at[sidxb], rowsb, semb)
            i3.wait()
            i4.wait()
            ga.wait()
            pltpu.sync_copy(rowsa, acc_sh.at[didxa], add=True)
            gb.wait()
            pltpu.sync_copy(rowsb, acc_sh.at[didxb], add=True)
            return carry

        return group

    for pi in range(2):
        cv = cbuf[pi, :]
        n = cv[0]
        nblk = (n + 2 * EPG - 1) // (2 * EPG)
        lax.fori_loop(0, nblk, group_for(2 * s + pi), None)

    plsc.subcore_barrier()
    for q in range(7):
        lo = s * STRIPE + q * EPG
        pltpu.sync_copy(acc_sh.at[pl.ds(lo, EPG)], rowsa)
        pltpu.sync_copy(w2_hbm.at[pl.ds(c * HALF + lo, EPG)], wbuf)

        def scale(g, carry):
            wv = wbuf[pl.ds(g * 16, 16)]
            for j in range(16):
                r = g * 16 + j
                for k in range(4):
                    rowsa[r, pl.ds(k * 16, 16)] = (
                        rowsa[r, pl.ds(k * 16, 16)] * wv[j])
            return carry

        lax.fori_loop(0, EPG // 16, scale, None)
        pltpu.sync_copy(rowsa, z_hbm.at[pl.ds(c * HALF + lo, EPG)])


@functools.partial(
    pl.kernel,
    out_type=jax.ShapeDtypeStruct((B, D), f32),
    mesh=_mesh,
    compiler_params=pltpu.CompilerParams(use_tc_tiling_on_sc=False),
    scratch_types=[
        pltpu.VMEM((B // 32,), i32),
        pltpu.VMEM((B // 32, D), f32),
        pltpu.SemaphoreType.DMA,
    ],
)
def _gather_users(acc_hbm, users_hbm, ue_hbm, uidx, urows, sem):
    wid = lax.axis_index("s") * 2 + lax.axis_index("c")
    base = wid * (B // 32)
    pltpu.sync_copy(users_hbm.at[pl.ds(base, B // 32)], uidx)
    pltpu.async_copy(acc_hbm.at[uidx], urows, sem).wait()
    pltpu.sync_copy(urows, ue_hbm.at[pl.ds(base, B // 32)])


def _dinv_body(dego_ref, degi_ref, dinvo_ref, w2_ref, rdo_ref):
    dgo = jnp.maximum(dego_ref[...], 1.0)
    dinvo = lax.rsqrt(dgo)
    dinvi = lax.rsqrt(jnp.maximum(degi_ref[...], 1.0))
    dinvo_ref[...] = dinvo
    w2_ref[...] = dinvo * dinvi
    rdo_ref[...] = jnp.sqrt(dgo)


_dinv = pl.pallas_call(
    _dinv_body,
    grid=(NP // STRIPE,),
    in_specs=[pl.BlockSpec((STRIPE, 16), lambda i: (i, 0)) for _ in range(2)],
    out_specs=tuple(pl.BlockSpec((STRIPE, 16), lambda i: (i, 0)) for _ in range(3)),
    out_shape=tuple(jax.ShapeDtypeStruct((NP, 16), f32) for _ in range(3)),
)


def _scale0_body(x_ref, dinvo_ref, y_ref):
    y_ref[...] = x_ref[...] * dinvo_ref[...][:, :1]


_scale0 = pl.pallas_call(
    _scale0_body,
    grid=(NP // STRIPE,),
    in_specs=[
        pl.BlockSpec((STRIPE, D), lambda i: (i, 0)),
        pl.BlockSpec((STRIPE, 16), lambda i: (i, 0)),
    ],
    out_specs=pl.BlockSpec((STRIPE, D), lambda i: (i, 0)),
    out_shape=jax.ShapeDtypeStruct((NP, D), f32),
)


def _finish_body(x0_ref, y1_ref, y2_ref, y3_ref, rdo_ref, acc_ref):
    ysum = y1_ref[...] + y2_ref[...] + y3_ref[...]
    acc_ref[...] = x0_ref[...] + ysum * rdo_ref[...][:, :1]


_finish = pl.pallas_call(
    _finish_body,
    grid=(NP // STRIPE,),
    in_specs=[
        pl.BlockSpec((STRIPE, D), lambda i: (i, 0)),
        pl.BlockSpec((STRIPE, D), lambda i: (i, 0)),
        pl.BlockSpec((STRIPE, D), lambda i: (i, 0)),
        pl.BlockSpec((STRIPE, D), lambda i: (i, 0)),
        pl.BlockSpec((STRIPE, 16), lambda i: (i, 0)),
    ],
    out_specs=pl.BlockSpec((STRIPE, D), lambda i: (i, 0)),
    out_shape=jax.ShapeDtypeStruct((NP, D), f32),
)


BK = 512
NBK = 49


def _matmul_body(ue_ref, items_ref, out_ref):
    out_ref[...] = lax.dot_general(
        ue_ref[...], items_ref[...],
        (((1,), (1,)), ((), ())),
        preferred_element_type=f32,
    ) * (1.0 / 16.0)


_matmul = pl.pallas_call(
    _matmul_body,
    grid=(NBK,),
    in_specs=[
        pl.BlockSpec((B, D), lambda j: (0, 0)),
        pl.BlockSpec((BK, D), lambda j: (j + HALF // BK, 0)),
    ],
    out_specs=pl.BlockSpec((B, BK), lambda j: (0, j)),
    out_shape=jax.ShapeDtypeStruct((B, NU), f32),
)


def kernel(users, edge_index, user_embedding, item_embedding):
    src = edge_index[0].astype(i32)
    dst = edge_index[1].astype(i32)
    padn = EP - E
    fill = (lax.iota(i32, padn) * 131) % N
    srcp = jnp.concatenate([src, fill]).reshape(ROWS, GB)
    dstp = jnp.concatenate([dst, jnp.full((padn,), N, i32)]).reshape(ROWS, GB)

    dl0, dl1, sl0, sl1 = _prep(srcp, dstp)
    dloc4 = jnp.stack([dl0, dl1]).reshape(2, NT, EPT)
    srcl4 = jnp.stack([sl0, sl1]).reshape(2, NT, EPT)

    psrc, pdst, pcnt = _part(srcp.reshape(32, EPT2), dstp.reshape(32, EPT2))
    dego, degi = _deg(srcl4, dloc4)
    dinvo, w2, rdo = _dinv(dego, degi)
    w2flat = w2[:, 0]

    zpad = jnp.zeros((HALF - NU, D), f32)
    x0 = jnp.concatenate([user_embedding, zpad, item_embedding, zpad], axis=0)
    y1 = _spmm(_scale0(x0, dinvo), psrc, pdst, pcnt, w2flat)
    y2 = _spmm(y1, psrc, pdst, pcnt, w2flat)
    y3 = _spmm(y2, psrc, pdst, pcnt, w2flat)
    acc = _finish(x0, y1, y2, y3, rdo)

    ue = _gather_users(acc, users.astype(i32))
    return _matmul(ue, acc)


# speedup vs baseline: 17.0878x; 1.0390x over previous
import functools

import jax
import jax.numpy as jnp
from jax import lax
from jax.experimental import pallas as pl
from jax.experimental.pallas import tpu as pltpu
from jax.experimental.pallas import tpu_sc as plsc

f32 = jnp.float32
i32 = jnp.int32

NU = 25000
N = 50000
D = 64
E = 800000
B = 1024
HALF = 25088
NP = 2 * HALF
NT = 16
GB = 128
GPT = 392
GRP = 8
NGRP = 49
EP = NT * GPT * GB
ROWS = NT * GPT
STRIPE = HALF // NT
NTRASH = 80

_mesh = plsc.VectorSubcoreMesh(core_axis_name="c", subcore_axis_name="s")


def _prep_body(src_ref, dst_ref, dl0_ref, dl1_ref, sl0_ref, sl1_ref):
    src = src_ref[...]
    dst = dst_ref[...]
    pid = pl.program_id(0)
    r = pid * GPT + lax.broadcasted_iota(i32, (GPT, GB), 0)
    l = lax.broadcasted_iota(i32, (GPT, GB), 1)
    toff = NU + ((r * GB + l) % NTRASH)
    dl0_ref[...] = jnp.where(dst < NU, dst, toff)
    dl1_ref[...] = jnp.where((dst >= NU) & (dst < N), dst - NU, toff)
    sl0_ref[...] = jnp.where(src < NU, src, toff)
    sl1_ref[...] = jnp.where((src >= NU) & (src < N), src - NU, toff)


_prep = pl.pallas_call(
    _prep_body,
    grid=(NT,),
    in_specs=[pl.BlockSpec((GPT, GB), lambda i: (i, 0)) for _ in range(2)],
    out_specs=tuple(pl.BlockSpec((GPT, GB), lambda i: (i, 0)) for _ in range(4)),
    out_shape=tuple(jax.ShapeDtypeStruct((ROWS, GB), i32) for _ in range(4)),
)


EPG = 224
NPAIR = 112
EPT = GPT * GB
DG = 1024
NDG = EPT // DG


@functools.partial(
    pl.kernel,
    out_type=(
        jax.ShapeDtypeStruct((NP, 16), f32),
        jax.ShapeDtypeStruct((NP, 16), f32),
    ),
    mesh=_mesh,
    compiler_params=pltpu.CompilerParams(use_tc_tiling_on_sc=False),
    scratch_types=[
        pltpu.VMEM((DG,), i32),
        pltpu.VMEM((DG,), i32),
        pltpu.VMEM((DG, 16), f32),
        pltpu.VMEM((STRIPE, 16), f32),
        pltpu.VMEM_SHARED((HALF, 16), f32),
        pltpu.VMEM_SHARED((HALF, 16), f32),
        pltpu.SemaphoreType.DMA,
    ],
)
def _deg(srcl_hbm, dloc_hbm, dego_hbm, degi_hbm, sidx, didx, ones_v, zv,
         dego_sh, degi_sh, semi):
    c = lax.axis_index("c")
    s = lax.axis_index("s")

    def fill(i, carry):
        ones_v[i, :] = jnp.ones((16,), f32)
        return carry

    lax.fori_loop(0, DG, fill, None)

    def fillz(i, carry):
        zv[i, :] = jnp.zeros((16,), f32)
        return carry

    lax.fori_loop(0, STRIPE, fillz, None)
    pltpu.sync_copy(zv, dego_sh.at[pl.ds(s * STRIPE, STRIPE)])
    pltpu.sync_copy(zv, degi_sh.at[pl.ds(s * STRIPE, STRIPE)])
    plsc.subcore_barrier()

    def group(h, carry):
        base = h * DG
        ca = pltpu.async_copy(srcl_hbm.at[c, s, pl.ds(base, DG)], sidx, semi)
        cb = pltpu.async_copy(dloc_hbm.at[c, s, pl.ds(base, DG)], didx, semi)
        ca.wait()
        cb.wait()
        pltpu.sync_copy(ones_v, dego_sh.at[sidx], add=True)
        pltpu.sync_copy(ones_v, degi_sh.at[didx], add=True)
        return carry

    lax.fori_loop(0, NDG, group, None)
    plsc.subcore_barrier()
    dst_lo = c * HALF + s * STRIPE
    pltpu.sync_copy(dego_sh.at[pl.ds(s * STRIPE, STRIPE)],
                    dego_hbm.at[pl.ds(dst_lo, STRIPE)])
    pltpu.sync_copy(degi_sh.at[pl.ds(s * STRIPE, STRIPE)],
                    degi_hbm.at[pl.ds(dst_lo, STRIPE)])


EPT2 = EP // 32
CAP = EPT2 + 2 * EPG
PSTG = 6272


@functools.partial(
    pl.kernel,
    out_type=(
        jax.ShapeDtypeStruct((2, 32, CAP), i32),
        jax.ShapeDtypeStruct((2, 32, CAP), i32),
        jax.ShapeDtypeStruct((2, 32, 16), i32),
    ),
    mesh=_mesh,
    compiler_params=pltpu.CompilerParams(use_tc_tiling_on_sc=False,
                                         needs_layout_passes=False),
    scratch_types=[
        pltpu.VMEM((PSTG,), i32),
        pltpu.VMEM((PSTG,), i32),
        pltpu.VMEM((CAP,), i32),
        pltpu.VMEM((CAP,), i32),
        pltpu.VMEM((CAP,), i32),
        pltpu.VMEM((CAP,), i32),
        pltpu.VMEM((16,), i32),
    ],
)
def _part(srcp_hbm, dstp_hbm, psrc_hbm, pdst_hbm, pcnt_hbm,
          sbuf, dbuf, sl0, dl0, sl1, dl1, cntv):
    c = lax.axis_index("c")
    s = lax.axis_index("s")
    w = c * NT + s
    lane = lax.iota(i32, 16)

    def outer(g, carry):
        pltpu.sync_copy(srcp_hbm.at[w, pl.ds(g * PSTG, PSTG)], sbuf)
        pltpu.sync_copy(dstp_hbm.at[w, pl.ds(g * PSTG, PSTG)], dbuf)

        def inner(i, carry2):
            c0, c1 = carry2
            sv = sbuf[pl.ds(i * 16, 16)]
            dv = dbuf[pl.ds(i * 16, 16)]
            sg = sv + 88 * (sv >= NU).astype(i32)
            m0 = dv < NU
            m1i = (dv >= NU) & (dv < N)
            m0i = m0.astype(i32)
            m1 = m1i.astype(i32)
            pos0 = c0 + plsc.cumsum(m0i) - m0i
            pos1 = c1 + plsc.cumsum(m1) - m1
            idx0 = jnp.where(m0, pos0, CAP - 16 + lane)
            idx1 = jnp.where(m1i, pos1, CAP - 16 + lane)
            plsc.store_scatter(sl0, [idx0], sg)
            plsc.store_scatter(dl0, [idx0], dv)
            plsc.store_scatter(sl1, [idx1], sg)
            plsc.store_scatter(dl1, [idx1], dv - NU)
            return (c0 + plsc.all_reduce_population_count(m0),
                    c1 + plsc.all_reduce_population_count(m1i))

        return lax.fori_loop(0, PSTG // 16, inner, carry)

    zv16 = jnp.zeros((16,), i32)
    cnt0, cnt1 = lax.fori_loop(0, EPT2 // PSTG, outer, (zv16, zv16))

    trash = NU + lane
    for k in range(2 * EPG // 16):
        off = k * 16 + lane
        plsc.store_scatter(sl0, [cnt0 + off], lane)
        plsc.store_scatter(dl0, [cnt0 + off], trash)
        plsc.store_scatter(sl1, [cnt1 + off], lane)
        plsc.store_scatter(dl1, [cnt1 + off], trash)
    pltpu.sync_copy(sl0, psrc_hbm.at[0, w])
    pltpu.sync_copy(dl0, pdst_hbm.at[0, w])
    pltpu.sync_copy(sl1, psrc_hbm.at[1, w])
    pltpu.sync_copy(dl1, pdst_hbm.at[1, w])
    cntv[pl.ds(0, 16)] = cnt0
    pltpu.sync_copy(cntv, pcnt_hbm.at[0, w])
    cntv[pl.ds(0, 16)] = cnt1
    pltpu.sync_copy(cntv, pcnt_hbm.at[1, w])


@functools.partial(
    pl.kernel,
    out_type=jax.ShapeDtypeStruct((NP, D), f32),
    mesh=_mesh,
    compiler_params=pltpu.CompilerParams(use_tc_tiling_on_sc=False),
    scratch_types=[
        pltpu.VMEM((EPG,), i32),
        pltpu.VMEM((EPG,), i32),
        pltpu.VMEM((EPG,), i32),
        pltpu.VMEM((EPG,), i32),
        pltpu.VMEM((2, 16), i32),
        pltpu.VMEM((EPG,), f32),
        pltpu.VMEM((EPG,), f32),
        pltpu.VMEM((EPG, D), f32),
        pltpu.VMEM((EPG, D), f32),
        pltpu.VMEM_SHARED((HALF, D), f32),
        pltpu.SemaphoreType.DMA,
        pltpu.SemaphoreType.DMA,
        pltpu.SemaphoreType.DMA,
        pltpu.SemaphoreType.DMA,
    ],
)
def _spmm(y_hbm, psrc_hbm, pdst_hbm, pcnt_hbm, w2_hbm, z_hbm, sidxa, sidxb,
          didxa, didxb, cbuf, wbuf, wbuf2, rowsa, rowsb, acc_sh, sema, semb,
          semi, semo):
    c = lax.axis_index("c")
    s = lax.axis_index("s")

    def fillz(i, carry):
        for k in range(4):
            rowsa[i, pl.ds(k * 16, 16)] = jnp.zeros((16,), f32)
        return carry

    lax.fori_loop(0, EPG, fillz, None)
    for q in range(7):
        pltpu.sync_copy(rowsa, acc_sh.at[pl.ds(s * STRIPE + q * EPG, EPG)])
    plsc.subcore_barrier()

    pltpu.sync_copy(pcnt_hbm.at[c, pl.ds(2 * s, 2)], cbuf)

    def group_for(p):
        def group(h, carry):
            base = h * 2 * EPG
            i1 = pltpu.async_copy(psrc_hbm.at[c, p, pl.ds(base, EPG)],
                                  sidxa, semi)
            i2 = pltpu.async_copy(psrc_hbm.at[c, p, pl.ds(base + EPG, EPG)],
                                  sidxb, semi)
            i3 = pltpu.async_copy(pdst_hbm.at[c, p, pl.ds(base, EPG)],
                                  didxa, semi)
            i4 = pltpu.async_copy(pdst_hbm.at[c, p, pl.ds(base + EPG, EPG)],
                                  didxb, semi)
            i1.wait()
            ga = pltpu.async_copy(y_hbm.at[sidxa], rowsa, sema)
            i2.wait()
            gb = pltpu.async_copy(y_hbm.at[sidxb], rowsb, semb)
            i3.wait()
            i4.wait()
            ga.wait()
            pltpu.sync_copy(rowsa, acc_sh.at[didxa], add=True)
            gb.wait()
            pltpu.sync_copy(rowsb, acc_sh.at[didxb], add=True)
            return carry

        return group

    for pi in range(2):
        cv = cbuf[pi, :]
        n = cv[0]
        nblk = (n + 2 * EPG - 1) // (2 * EPG)
        lax.fori_loop(0, nblk, group_for(2 * s + pi), None)

    plsc.subcore_barrier()

    def scale_of(buf, wb):
        def scale(g, carry):
            wv = wb[pl.ds(g * 16, 16)]
            for j in range(16):
                r = g * 16 + j
                for k in range(4):
                    buf[r, pl.ds(k * 16, 16)] = (
                        buf[r, pl.ds(k * 16, 16)] * wv[j])
            return carry
        return scale

    bufs = [(rowsa, wbuf, sema), (rowsb, wbuf2, semb)]
    h_in = {}
    h_out = {}

    def fire_in(q):
        buf, wb, sem = bufs[q % 2]
        lo = s * STRIPE + q * EPG
        h_in[q] = (
            pltpu.async_copy(acc_sh.at[pl.ds(lo, EPG)], buf, sem),
            pltpu.async_copy(w2_hbm.at[pl.ds(c * HALF + lo, EPG)], wb, semi),
        )

    fire_in(0)
    for q in range(7):
        buf, wb, _ = bufs[q % 2]
        for hh in h_in[q]:
            hh.wait()
        if q + 1 < 7:
            if q - 1 >= 0:
                h_out[q - 1].wait()
            fire_in(q + 1)
        lax.fori_loop(0, EPG // 16, scale_of(buf, wb), None)
        lo = s * STRIPE + q * EPG
        h_out[q] = pltpu.async_copy(
            buf, z_hbm.at[pl.ds(c * HALF + lo, EPG)], semo)
    h_out[5].wait()
    h_out[6].wait()


@functools.partial(
    pl.kernel,
    out_type=jax.ShapeDtypeStruct((B, D), f32),
    mesh=_mesh,
    compiler_params=pltpu.CompilerParams(use_tc_tiling_on_sc=False),
    scratch_types=[
        pltpu.VMEM((B // 32,), i32),
        pltpu.VMEM((B // 32, D), f32),
        pltpu.SemaphoreType.DMA,
    ],
)
def _gather_users(acc_hbm, users_hbm, ue_hbm, uidx, urows, sem):
    wid = lax.axis_index("s") * 2 + lax.axis_index("c")
    base = wid * (B // 32)
    pltpu.sync_copy(users_hbm.at[pl.ds(base, B // 32)], uidx)
    pltpu.async_copy(acc_hbm.at[uidx], urows, sem).wait()
    pltpu.sync_copy(urows, ue_hbm.at[pl.ds(base, B // 32)])


def _dinv_body(dego_ref, degi_ref, x0_ref, y0_ref, w2_ref, rdo_ref):
    dgo = jnp.maximum(dego_ref[...], 1.0)
    dinvo = lax.rsqrt(dgo)
    dinvi = lax.rsqrt(jnp.maximum(degi_ref[...], 1.0))
    y0_ref[...] = x0_ref[...] * dinvo[:, :1]
    w2_ref[...] = dinvo * dinvi
    rdo_ref[...] = jnp.sqrt(dgo)


_dinv = pl.pallas_call(
    _dinv_body,
    grid=(NP // STRIPE,),
    in_specs=[pl.BlockSpec((STRIPE, 16), lambda i: (i, 0)) for _ in range(2)]
    + [pl.BlockSpec((STRIPE, D), lambda i: (i, 0))],
    out_specs=(
        pl.BlockSpec((STRIPE, D), lambda i: (i, 0)),
        pl.BlockSpec((STRIPE, 16), lambda i: (i, 0)),
        pl.BlockSpec((STRIPE, 16), lambda i: (i, 0)),
    ),
    out_shape=(
        jax.ShapeDtypeStruct((NP, D), f32),
        jax.ShapeDtypeStruct((NP, 16), f32),
        jax.ShapeDtypeStruct((NP, 16), f32),
    ),
)


def _finish_body(x0_ref, y1_ref, y2_ref, y3_ref, rdo_ref, acc_ref):
    ysum = y1_ref[...] + y2_ref[...] + y3_ref[...]
    acc_ref[...] = x0_ref[...] + ysum * rdo_ref[...][:, :1]


_finish = pl.pallas_call(
    _finish_body,
    grid=(NP // STRIPE,),
    in_specs=[
        pl.BlockSpec((STRIPE, D), lambda i: (i, 0)),
        pl.BlockSpec((STRIPE, D), lambda i: (i, 0)),
        pl.BlockSpec((STRIPE, D), lambda i: (i, 0)),
        pl.BlockSpec((STRIPE, D), lambda i: (i, 0)),
        pl.BlockSpec((STRIPE, 16), lambda i: (i, 0)),
    ],
    out_specs=pl.BlockSpec((STRIPE, D), lambda i: (i, 0)),
    out_shape=jax.ShapeDtypeStruct((NP, D), f32),
)


BK = 512
NBK = 49


def _matmul_body(ue_ref, items_ref, out_ref):
    out_ref[...] = lax.dot_general(
        ue_ref[...], items_ref[...],
        (((1,), (1,)), ((), ())),
        preferred_element_type=f32,
    ) * (1.0 / 16.0)


_matmul = pl.pallas_call(
    _matmul_body,
    grid=(NBK,),
    in_specs=[
        pl.BlockSpec((B, D), lambda j: (0, 0)),
        pl.BlockSpec((BK, D), lambda j: (j + HALF // BK, 0)),
    ],
    out_specs=pl.BlockSpec((B, BK), lambda j: (0, j)),
    out_shape=jax.ShapeDtypeStruct((B, NU), f32),
)


def kernel(users, edge_index, user_embedding, item_embedding):
    src = edge_index[0].astype(i32)
    dst = edge_index[1].astype(i32)
    padn = EP - E
    fill = (lax.iota(i32, padn) * 131) % N
    srcp = jnp.concatenate([src, fill]).reshape(ROWS, GB)
    dstp = jnp.concatenate([dst, jnp.full((padn,), N, i32)]).reshape(ROWS, GB)

    dl0, dl1, sl0, sl1 = _prep(srcp, dstp)
    dloc4 = jnp.stack([dl0, dl1]).reshape(2, NT, EPT)
    srcl4 = jnp.stack([sl0, sl1]).reshape(2, NT, EPT)

    psrc, pdst, pcnt = _part(srcp.reshape(32, EPT2), dstp.reshape(32, EPT2))
    dego, degi = _deg(srcl4, dloc4)

    zpad = jnp.zeros((HALF - NU, D), f32)
    x0 = jnp.concatenate([user_embedding, zpad, item_embedding, zpad], axis=0)
    y0, w2, rdo = _dinv(dego, degi, x0)
    w2flat = w2[:, 0]
    y1 = _spmm(y0, psrc, pdst, pcnt, w2flat)
    y2 = _spmm(y1, psrc, pdst, pcnt, w2flat)
    y3 = _spmm(y2, psrc, pdst, pcnt, w2flat)
    acc = _finish(x0, y1, y2, y3, rdo)

    ue = _gather_users(acc, users.astype(i32))
    return _matmul(ue, acc)


# speedup vs baseline: 17.7134x; 1.0366x over previous
import functools

import jax
import jax.numpy as jnp
from jax import lax
from jax.experimental import pallas as pl
from jax.experimental.pallas import tpu as pltpu
from jax.experimental.pallas import tpu_sc as plsc

f32 = jnp.float32
i32 = jnp.int32

NU = 25000
N = 50000
D = 64
E = 800000
B = 1024
HALF = 25088
NP = 2 * HALF
NT = 16
GB = 128
GPT = 392
GRP = 8
NGRP = 49
EP = NT * GPT * GB
ROWS = NT * GPT
STRIPE = HALF // NT
NTRASH = 80

_mesh = plsc.VectorSubcoreMesh(core_axis_name="c", subcore_axis_name="s")


EPG = 224
NPAIR = 112
EPT = GPT * GB
DG = 1024
NDG = EPT // DG


@functools.partial(
    pl.kernel,
    out_type=(
        jax.ShapeDtypeStruct((NP, 16), f32),
        jax.ShapeDtypeStruct((NP, 16), f32),
    ),
    mesh=_mesh,
    compiler_params=pltpu.CompilerParams(use_tc_tiling_on_sc=False),
    scratch_types=[
        pltpu.VMEM((DG,), i32),
        pltpu.VMEM((DG,), i32),
        pltpu.VMEM((DG,), i32),
        pltpu.VMEM((DG,), i32),
        pltpu.VMEM((DG, 16), f32),
        pltpu.VMEM((STRIPE, 16), f32),
        pltpu.VMEM_SHARED((HALF, 16), f32),
        pltpu.VMEM_SHARED((HALF, 16), f32),
        pltpu.SemaphoreType.DMA,
    ],
)
def _deg(srcp_hbm, dstp_hbm, dego_hbm, degi_hbm, sraw, draw, sidx, didx,
         ones_v, zv, dego_sh, degi_sh, semi):
    c = lax.axis_index("c")
    s = lax.axis_index("s")
    lane = lax.iota(i32, 16)
    base = c * NU

    def fill(i, carry):
        ones_v[i, :] = jnp.ones((16,), f32)
        return carry

    lax.fori_loop(0, DG, fill, None)

    def fillz(i, carry):
        zv[i, :] = jnp.zeros((16,), f32)
        return carry

    lax.fori_loop(0, STRIPE, fillz, None)
    pltpu.sync_copy(zv, dego_sh.at[pl.ds(s * STRIPE, STRIPE)])
    pltpu.sync_copy(zv, degi_sh.at[pl.ds(s * STRIPE, STRIPE)])
    plsc.subcore_barrier()

    def group(h, carry):
        lo = h * DG
        ca = pltpu.async_copy(srcp_hbm.at[s, pl.ds(lo, DG)], sraw, semi)
        cb = pltpu.async_copy(dstp_hbm.at[s, pl.ds(lo, DG)], draw, semi)
        ca.wait()
        cb.wait()

        def chunk(k, carry2):
            sv = sraw[pl.ds(k * 16, 16)]
            dv = draw[pl.ds(k * 16, 16)]
            tr = NU + jnp.mod(lane + k, NTRASH)
            sidx[pl.ds(k * 16, 16)] = jnp.where(
                (sv >= base) & (sv < base + NU), sv - base, tr)
            didx[pl.ds(k * 16, 16)] = jnp.where(
                (dv >= base) & (dv < base + NU), dv - base, tr)
            return carry2

        lax.fori_loop(0, DG // 16, chunk, None)
        pltpu.sync_copy(ones_v, dego_sh.at[sidx], add=True)
        pltpu.sync_copy(ones_v, degi_sh.at[didx], add=True)
        return carry

    lax.fori_loop(0, NDG, group, None)
    plsc.subcore_barrier()
    dst_lo = c * HALF + s * STRIPE
    pltpu.sync_copy(dego_sh.at[pl.ds(s * STRIPE, STRIPE)],
                    dego_hbm.at[pl.ds(dst_lo, STRIPE)])
    pltpu.sync_copy(degi_sh.at[pl.ds(s * STRIPE, STRIPE)],
                    degi_hbm.at[pl.ds(dst_lo, STRIPE)])


EPT2 = EP // 32
CAP = EPT2 + 2 * EPG
PSTG = 6272


@functools.partial(
    pl.kernel,
    out_type=(
        jax.ShapeDtypeStruct((2, 32, CAP), i32),
        jax.ShapeDtypeStruct((2, 32, CAP), i32),
        jax.ShapeDtypeStruct((2, 32, 16), i32),
    ),
    mesh=_mesh,
    compiler_params=pltpu.CompilerParams(use_tc_tiling_on_sc=False,
                                         needs_layout_passes=False),
    scratch_types=[
        pltpu.VMEM((PSTG,), i32),
        pltpu.VMEM((PSTG,), i32),
        pltpu.VMEM((CAP,), i32),
        pltpu.VMEM((CAP,), i32),
        pltpu.VMEM((CAP,), i32),
        pltpu.VMEM((CAP,), i32),
        pltpu.VMEM((16,), i32),
    ],
)
def _part(srcp_hbm, dstp_hbm, psrc_hbm, pdst_hbm, pcnt_hbm,
          sbuf, dbuf, sl0, dl0, sl1, dl1, cntv):
    c = lax.axis_index("c")
    s = lax.axis_index("s")
    w = c * NT + s
    lane = lax.iota(i32, 16)

    def outer(g, carry):
        pltpu.sync_copy(srcp_hbm.at[w, pl.ds(g * PSTG, PSTG)], sbuf)
        pltpu.sync_copy(dstp_hbm.at[w, pl.ds(g * PSTG, PSTG)], dbuf)

        def inner(i, carry2):
            c0, c1 = carry2
            sv = sbuf[pl.ds(i * 16, 16)]
            dv = dbuf[pl.ds(i * 16, 16)]
            sg = sv + 88 * (sv >= NU).astype(i32)
            m0 = dv < NU
            m1i = (dv >= NU) & (dv < N)
            m0i = m0.astype(i32)
            m1 = m1i.astype(i32)
            pos0 = c0 + plsc.cumsum(m0i) - m0i
            pos1 = c1 + plsc.cumsum(m1) - m1
            idx0 = jnp.where(m0, pos0, CAP - 16 + lane)
            idx1 = jnp.where(m1i, pos1, CAP - 16 + lane)
            plsc.store_scatter(sl0, [idx0], sg)
            plsc.store_scatter(dl0, [idx0], dv)
            plsc.store_scatter(sl1, [idx1], sg)
            plsc.store_scatter(dl1, [idx1], dv - NU)
            return (c0 + plsc.all_reduce_population_count(m0),
                    c1 + plsc.all_reduce_population_count(m1i))

        return lax.fori_loop(0, PSTG // 16, inner, carry)

    zv16 = jnp.zeros((16,), i32)
    cnt0, cnt1 = lax.fori_loop(0, EPT2 // PSTG, outer, (zv16, zv16))

    trash = NU + lane
    for k in range(2 * EPG // 16):
        off = k * 16 + lane
        plsc.store_scatter(sl0, [cnt0 + off], lane)
        plsc.store_scatter(dl0, [cnt0 + off], trash)
        plsc.store_scatter(sl1, [cnt1 + off], lane)
        plsc.store_scatter(dl1, [cnt1 + off], trash)
    pltpu.sync_copy(sl0, psrc_hbm.at[0, w])
    pltpu.sync_copy(dl0, pdst_hbm.at[0, w])
    pltpu.sync_copy(sl1, psrc_hbm.at[1, w])
    pltpu.sync_copy(dl1, pdst_hbm.at[1, w])
    cntv[pl.ds(0, 16)] = cnt0
    pltpu.sync_copy(cntv, pcnt_hbm.at[0, w])
    cntv[pl.ds(0, 16)] = cnt1
    pltpu.sync_copy(cntv, pcnt_hbm.at[1, w])


@functools.partial(
    pl.kernel,
    out_type=jax.ShapeDtypeStruct((NP, D), f32),
    mesh=_mesh,
    compiler_params=pltpu.CompilerParams(use_tc_tiling_on_sc=False),
    scratch_types=[
        pltpu.VMEM((EPG,), i32),
        pltpu.VMEM((EPG,), i32),
        pltpu.VMEM((EPG,), i32),
        pltpu.VMEM((EPG,), i32),
        pltpu.VMEM((2, 16), i32),
        pltpu.VMEM((EPG,), f32),
        pltpu.VMEM((EPG,), f32),
        pltpu.VMEM((EPG, D), f32),
        pltpu.VMEM((EPG, D), f32),
        pltpu.VMEM_SHARED((HALF, D), f32),
        pltpu.SemaphoreType.DMA,
        pltpu.SemaphoreType.DMA,
        pltpu.SemaphoreType.DMA,
        pltpu.SemaphoreType.DMA,
    ],
)
def _spmm(y_hbm, psrc_hbm, pdst_hbm, pcnt_hbm, w2_hbm, z_hbm, sidxa, sidxb,
          didxa, didxb, cbuf, wbuf, wbuf2, rowsa, rowsb, acc_sh, sema, semb,
          semi, semo):
    c = lax.axis_index("c")
    s = lax.axis_index("s")

    def fillz(i, carry):
        for k in range(4):
            rowsa[i, pl.ds(k * 16, 16)] = jnp.zeros((16,), f32)
        return carry

    lax.fori_loop(0, EPG, fillz, None)
    for q in range(7):
        pltpu.sync_copy(rowsa, acc_sh.at[pl.ds(s * STRIPE + q * EPG, EPG)])
    plsc.subcore_barrier()

    pltpu.sync_copy(pcnt_hbm.at[c, pl.ds(2 * s, 2)], cbuf)

    def group_for(p):
        def group(h, carry):
            base = h * 2 * EPG
            i1 = pltpu.async_copy(psrc_hbm.at[c, p, pl.ds(base, EPG)],
                                  sidxa, semi)
            i2 = pltpu.async_copy(psrc_hbm.at[c, p, pl.ds(base + EPG, EPG)],
                                  sidxb, semi)
            i3 = pltpu.async_copy(pdst_hbm.at[c, p, pl.ds(base, EPG)],
                                  didxa, semi)
            i4 = pltpu.async_copy(pdst_hbm.at[c, p, pl.ds(base + EPG, EPG)],
                                  didxb, semi)
            i1.wait()
            ga = pltpu.async_copy(y_hbm.at[sidxa], rowsa, sema)
            i2.wait()
            gb = pltpu.async_copy(y_hbm.at[sidxb], rowsb, semb)
            i3.wait()
            i4.wait()
            ga.wait()
            pltpu.sync_copy(rowsa, acc_sh.at[didxa], add=True)
            gb.wait()
            pltpu.sync_copy(rowsb, acc_sh.at[didxb], add=True)
            return carry

        return group

    for pi in range(2):
        cv = cbuf[pi, :]
        n = cv[0]
        nblk = (n + 2 * EPG - 1) // (2 * EPG)
        lax.fori_loop(0, nblk, group_for(2 * s + pi), None)

    plsc.subcore_barrier()

    def scale_of(buf, wb):
        def scale(g, carry):
            wv = wb[pl.ds(g * 16, 16)]
            for j in range(16):
                r = g * 16 + j
                for k in range(4):
                    buf[r, pl.ds(k * 16, 16)] = (
                        buf[r, pl.ds(k * 16, 16)] * wv[j])
            return carry
        return scale

    bufs = [(rowsa, wbuf, sema), (rowsb, wbuf2, semb)]
    h_in = {}
    h_out = {}

    def fire_in(q):
        buf, wb, sem = bufs[q % 2]
        lo = s * STRIPE + q * EPG
        h_in[q] = (
            pltpu.async_copy(acc_sh.at[pl.ds(lo, EPG)], buf, sem),
            pltpu.async_copy(w2_hbm.at[pl.ds(c * HALF + lo, EPG)], wb, semi),
        )

    fire_in(0)
    for q in range(7):
        buf, wb, _ = bufs[q % 2]
        for hh in h_in[q]:
            hh.wait()
        if q + 1 < 7:
            if q - 1 >= 0:
                h_out[q - 1].wait()
            fire_in(q + 1)
        lax.fori_loop(0, EPG // 16, scale_of(buf, wb), None)
        lo = s * STRIPE + q * EPG
        h_out[q] = pltpu.async_copy(
            buf, z_hbm.at[pl.ds(c * HALF + lo, EPG)], semo)
    h_out[5].wait()
    h_out[6].wait()


@functools.partial(
    pl.kernel,
    out_type=jax.ShapeDtypeStruct((B, D), f32),
    mesh=_mesh,
    compiler_params=pltpu.CompilerParams(use_tc_tiling_on_sc=False),
    scratch_types=[
        pltpu.VMEM((B // 32,), i32),
        pltpu.VMEM((B // 32, D), f32),
        pltpu.SemaphoreType.DMA,
    ],
)
def _gather_users(acc_hbm, users_hbm, ue_hbm, uidx, urows, sem):
    wid = lax.axis_index("s") * 2 + lax.axis_index("c")
    base = wid * (B // 32)
    pltpu.sync_copy(users_hbm.at[pl.ds(base, B // 32)], uidx)
    pltpu.async_copy(acc_hbm.at[uidx], urows, sem).wait()
    pltpu.sync_copy(urows, ue_hbm.at[pl.ds(base, B // 32)])


def _dinv_body(dego_ref, degi_ref, x0_ref, y0_ref, w2_ref, rdo_ref):
    dgo = jnp.maximum(dego_ref[...], 1.0)
    dinvo = lax.rsqrt(dgo)
    dinvi = lax.rsqrt(jnp.maximum(degi_ref[...], 1.0))
    y0_ref[...] = x0_ref[...] * dinvo[:, :1]
    w2_ref[...] = dinvo * dinvi
    rdo_ref[...] = jnp.sqrt(dgo)


_dinv = pl.pallas_call(
    _dinv_body,
    grid=(NP // STRIPE,),
    in_specs=[pl.BlockSpec((STRIPE, 16), lambda i: (i, 0)) for _ in range(2)]
    + [pl.BlockSpec((STRIPE, D), lambda i: (i, 0))],
    out_specs=(
        pl.BlockSpec((STRIPE, D), lambda i: (i, 0)),
        pl.BlockSpec((STRIPE, 16), lambda i: (i, 0)),
        pl.BlockSpec((STRIPE, 16), lambda i: (i, 0)),
    ),
    out_shape=(
        jax.ShapeDtypeStruct((NP, D), f32),
        jax.ShapeDtypeStruct((NP, 16), f32),
        jax.ShapeDtypeStruct((NP, 16), f32),
    ),
)


def _finish_body(x0_ref, y1_ref, y2_ref, y3_ref, rdo_ref, acc_ref):
    ysum = y1_ref[...] + y2_ref[...] + y3_ref[...]
    acc_ref[...] = x0_ref[...] + ysum * rdo_ref[...][:, :1]


_finish = pl.pallas_call(
    _finish_body,
    grid=(NP // STRIPE,),
    in_specs=[
        pl.BlockSpec((STRIPE, D), lambda i: (i, 0)),
        pl.BlockSpec((STRIPE, D), lambda i: (i, 0)),
        pl.BlockSpec((STRIPE, D), lambda i: (i, 0)),
        pl.BlockSpec((STRIPE, D), lambda i: (i, 0)),
        pl.BlockSpec((STRIPE, 16), lambda i: (i, 0)),
    ],
    out_specs=pl.BlockSpec((STRIPE, D), lambda i: (i, 0)),
    out_shape=jax.ShapeDtypeStruct((NP, D), f32),
)


BK = 512
NBK = 49


def _matmul_body(ue_ref, items_ref, out_ref):
    out_ref[...] = lax.dot_general(
        ue_ref[...], items_ref[...],
        (((1,), (1,)), ((), ())),
        preferred_element_type=f32,
    ) * (1.0 / 16.0)


_matmul = pl.pallas_call(
    _matmul_body,
    grid=(NBK,),
    in_specs=[
        pl.BlockSpec((B, D), lambda j: (0, 0)),
        pl.BlockSpec((BK, D), lambda j: (j + HALF // BK, 0)),
    ],
    out_specs=pl.BlockSpec((B, BK), lambda j: (0, j)),
    out_shape=jax.ShapeDtypeStruct((B, NU), f32),
)


def kernel(users, edge_index, user_embedding, item_embedding):
    src = edge_index[0].astype(i32)
    dst = edge_index[1].astype(i32)
    padn = EP - E
    fill = jnp.full((padn,), N, i32)
    srcp = jnp.concatenate([src, fill])
    dstp = jnp.concatenate([dst, fill])

    psrc, pdst, pcnt = _part(srcp.reshape(32, EPT2), dstp.reshape(32, EPT2))
    dego, degi = _deg(srcp.reshape(NT, EPT), dstp.reshape(NT, EPT))

    zpad = jnp.zeros((HALF - NU, D), f32)
    x0 = jnp.concatenate([user_embedding, zpad, item_embedding, zpad], axis=0)
    y0, w2, rdo = _dinv(dego, degi, x0)
    w2flat = w2[:, 0]
    y1 = _spmm(y0, psrc, pdst, pcnt, w2flat)
    y2 = _spmm(y1, psrc, pdst, pcnt, w2flat)
    y3 = _spmm(y2, psrc, pdst, pcnt, w2flat)
    acc = _finish(x0, y1, y2, y3, rdo)

    ue = _gather_users(acc, users.astype(i32))
    return _matmul(ue, acc)
